# Initial kernel scaffold; baseline (speedup 1.0000x reference)
import jax
import jax.numpy as jnp
from jax.experimental import pallas as pl


def kernel(x, edge_index, batch, sparse_mask, sm_weight, sm_bias, conv1_W, conv1_b, bn1_g, bn1_b, conv2_W, conv2_b, bn2_g, bn2_b, lin_W, lin_b):
    raise NotImplementedError("write your pallas kernel here")


# speedup vs baseline: 8.4101x; 8.4101x over previous
import functools

import jax
import jax.numpy as jnp
from jax import lax
from jax.experimental import pallas as pl
from jax.experimental.pallas import tpu as pltpu
from jax.experimental.pallas import tpu_sc as plsc

N = 10000
E = 320000
IN_F = 128
F = 64
HF = 32
NG = 8
NP = 10240
EP = 327680
NC, NS = 2, 16
EPT = EP // NS
ROWS_IT = 8
CH = ROWS_IT * 128
N_IT = EPT // CH
RPS = NP // NS
EPT_D = EP // (NC * NS)
N_IT_D = EPT_D // CH
RB = 256
GRID = NP // RB
EPS = 1e-5
f32 = jnp.float32


def _sc_mesh():
    return plsc.VectorSubcoreMesh(core_axis_name="c", subcore_axis_name="s")


def _deg_body(ones_hbm, z_hbm, src_hbm, out_hbm, srcv, onesv, stage, acc, sem):
    c = lax.axis_index("c")
    s = lax.axis_index("s")
    w = c * NS + s
    pltpu.sync_copy(z_hbm, stage)
    pltpu.sync_copy(stage, acc.at[pl.ds(s * RPS, RPS)])
    pltpu.sync_copy(ones_hbm, onesv)
    plsc.subcore_barrier()

    @pl.loop(0, N_IT_D)
    def _(it):
        pltpu.sync_copy(src_hbm.at[w, pl.ds(it * ROWS_IT, ROWS_IT)], srcv)
        for j in range(ROWS_IT):
            pltpu.sync_copy(onesv, acc.at[srcv.at[j]], add=True)

    plsc.subcore_barrier()
    pltpu.sync_copy(acc.at[pl.ds(s * RPS, RPS)], stage)
    pltpu.sync_copy(stage, out_hbm.at[c, pl.ds(s * RPS, RPS)])


@jax.jit
def _deg_pass(ones16, z16, srci):
    kern = pl.kernel(
        _deg_body,
        out_type=jax.ShapeDtypeStruct((NC, NP, 16), f32),
        mesh=_sc_mesh(),
        compiler_params=pltpu.CompilerParams(use_tc_tiling_on_sc=False),
        scratch_types=[
            pltpu.VMEM((ROWS_IT, 128), jnp.int32),
            pltpu.VMEM((128, 16), f32),
            pltpu.VMEM((RPS, 16), f32),
            pltpu.VMEM_SHARED((NP, 16), f32),
            pltpu.SemaphoreType.DMA,
        ],
    )
    return kern(ones16, z16, srci)


def _scat_body(tab_hbm, z_hbm, src2_hbm, dst_hbm, out_hbm,
               srcv, dstv, rows, stage, acc, sem):
    c = lax.axis_index("c")
    s = lax.axis_index("s")
    pltpu.sync_copy(z_hbm, stage)
    pltpu.sync_copy(stage, acc.at[pl.ds(s * RPS, RPS)])
    plsc.subcore_barrier()

    @pl.loop(0, N_IT)
    def _(it):
        pltpu.sync_copy(src2_hbm.at[c, s, pl.ds(it * ROWS_IT, ROWS_IT)], srcv)
        pltpu.sync_copy(dst_hbm.at[s, pl.ds(it * ROWS_IT, ROWS_IT)], dstv)
        cps = []
        for j in range(ROWS_IT):
            cps.append(pltpu.async_copy(
                tab_hbm.at[srcv.at[j]], rows.at[pl.ds(j * 128, 128)], sem))
        for cp in cps:
            cp.wait()
        for j in range(ROWS_IT):
            pltpu.sync_copy(rows.at[pl.ds(j * 128, 128)],
                            acc.at[dstv.at[j]], add=True)

    plsc.subcore_barrier()
    pltpu.sync_copy(acc.at[pl.ds(s * RPS, RPS)], stage)
    pltpu.sync_copy(stage, out_hbm.at[c, pl.ds(s * RPS, RPS)])


@jax.jit
def _scatter_pass(tab, z32, src2i, dsti):
    kern = pl.kernel(
        _scat_body,
        out_type=jax.ShapeDtypeStruct((NC, NP, HF), f32),
        mesh=_sc_mesh(),
        compiler_params=pltpu.CompilerParams(use_tc_tiling_on_sc=False),
        scratch_types=[
            pltpu.VMEM((ROWS_IT, 128), jnp.int32),
            pltpu.VMEM((ROWS_IT, 128), jnp.int32),
            pltpu.VMEM((CH, HF), f32),
            pltpu.VMEM((RPS, HF), f32),
            pltpu.VMEM_SHARED((NP, HF), f32),
            pltpu.SemaphoreType.DMA,
        ],
    )
    return kern(tab, z32, src2i, dsti)


def _dis(degp):
    deg = degp[0, :, 0:1] + degp[1, :, 0:1]
    return jnp.where(deg > 0, lax.rsqrt(jnp.maximum(deg, 1.0)), 0.0)


def _valid(i):
    row = lax.broadcasted_iota(jnp.int32, (RB, 1), 0) + i * RB
    return row < N


def _step1_body(x_ref, b1_ref, c1_ref, degp_ref, ga_ref, c0d_ref, d0_ref):
    i = pl.program_id(0)
    d = _dis(degp_ref[...])
    v = _valid(i)
    G = jnp.dot(x_ref[...], b1_ref[...], preferred_element_type=f32, precision=lax.Precision.HIGHEST) + c1_ref[...]
    ga_ref[0] = jnp.where(v, d * G[:, 0:HF], 0.0)
    ga_ref[1] = jnp.where(v, d * G[:, HF:F], 0.0)
    c0d_ref[...] = jnp.where(v, d * G[:, F:2 * F], 0.0)
    d0_ref[...] = jnp.where(v, G[:, 2 * F:3 * F], 0.0)


def _step3_body(c0d_ref, degp_ref, sp_ref, gc_ref):
    i = pl.program_id(0)
    d = _dis(degp_ref[...])
    v = _valid(i)
    d2 = d * d
    c0d = c0d_ref[...]
    gc_ref[0] = jnp.where(v, c0d[:, 0:HF] - d2 * sp_ref[0], 0.0)
    gc_ref[1] = jnp.where(v, c0d[:, HF:F] - d2 * sp_ref[1], 0.0)


def _step5_body(d0_ref, degp_ref, sp_ref, r_ref, st_ref):
    i = pl.program_id(0)
    d = _dis(degp_ref[...])
    v = _valid(i)
    d0 = d0_ref[...]
    rl = jnp.where(v, jnp.maximum(d0[:, 0:HF] - d * sp_ref[0], 0.0), 0.0)
    rr = jnp.where(v, jnp.maximum(d0[:, HF:F] - d * sp_ref[1], 0.0), 0.0)
    r_ref[:, 0:HF] = rl
    r_ref[:, HF:F] = rr

    @pl.when(i == 0)
    def _():
        st_ref[...] = jnp.zeros((8, F), f32)

    st_ref[0:1, 0:HF] = st_ref[0:1, 0:HF] + jnp.sum(rl, axis=0, keepdims=True)
    st_ref[0:1, HF:F] = st_ref[0:1, HF:F] + jnp.sum(rr, axis=0, keepdims=True)
    st_ref[1:2, 0:HF] = st_ref[1:2, 0:HF] + jnp.sum(rl * rl, axis=0, keepdims=True)
    st_ref[1:2, HF:F] = st_ref[1:2, HF:F] + jnp.sum(rr * rr, axis=0, keepdims=True)


def _step6_body(r_ref, b2_ref, b2r_ref, degp_ref, st_ref, g_ref, b_ref,
                ga2_ref, c2d_ref, d2o_ref):
    i = pl.program_id(0)
    d = _dis(degp_ref[...])
    v = _valid(i)
    m = st_ref[0:1, :] / N
    var = st_ref[1:2, :] / N - m * m
    sv = g_ref[...] * lax.rsqrt(var + EPS)
    tv = b_ref[...] - m * sv
    h1 = jnp.where(v, r_ref[...] * sv + tv, 0.0)
    G2 = jnp.dot(h1, b2_ref[...], preferred_element_type=f32, precision=lax.Precision.HIGHEST) + b2r_ref[...]
    ga2_ref[0] = jnp.where(v, d * G2[:, 0:HF], 0.0)
    ga2_ref[1] = jnp.where(v, d * G2[:, HF:F], 0.0)
    c2d_ref[...] = jnp.where(v, d * G2[:, F:2 * F], 0.0)
    d2o_ref[...] = jnp.where(v, G2[:, 2 * F:3 * F], 0.0)


def _step10_body(d2o_ref, degp_ref, sp_ref, bat_ref,
                 st_ref, ps_ref, pm_ref, pc_ref):
    i = pl.program_id(0)
    d = _dis(degp_ref[...])
    v = _valid(i)
    d2o = d2o_ref[...]
    rl = jnp.where(v, jnp.maximum(d2o[:, 0:HF] - d * sp_ref[0], 0.0), 0.0)
    rr = jnp.where(v, jnp.maximum(d2o[:, HF:F] - d * sp_ref[1], 0.0), 0.0)
    bat = bat_ref[...]

    @pl.when(i == 0)
    def _():
        st_ref[...] = jnp.zeros((8, F), f32)
        ps_ref[...] = jnp.zeros((8, F), f32)
        pm_ref[...] = jnp.full((8, F), -jnp.inf, f32)
        pc_ref[...] = jnp.zeros((8, F), f32)

    st_ref[0:1, 0:HF] = st_ref[0:1, 0:HF] + jnp.sum(rl, axis=0, keepdims=True)
    st_ref[0:1, HF:F] = st_ref[0:1, HF:F] + jnp.sum(rr, axis=0, keepdims=True)
    st_ref[1:2, 0:HF] = st_ref[1:2, 0:HF] + jnp.sum(rl * rl, axis=0, keepdims=True)
    st_ref[1:2, HF:F] = st_ref[1:2, HF:F] + jnp.sum(rr * rr, axis=0, keepdims=True)
    for g in range(NG):
        mg = bat == g
        ps_ref[g:g + 1, 0:HF] = ps_ref[g:g + 1, 0:HF] + jnp.sum(
            jnp.where(mg, rl, 0.0), axis=0, keepdims=True)
        ps_ref[g:g + 1, HF:F] = ps_ref[g:g + 1, HF:F] + jnp.sum(
            jnp.where(mg, rr, 0.0), axis=0, keepdims=True)
        pm_ref[g:g + 1, 0:HF] = jnp.maximum(
            pm_ref[g:g + 1, 0:HF],
            jnp.max(jnp.where(mg, rl, -jnp.inf), axis=0, keepdims=True))
        pm_ref[g:g + 1, HF:F] = jnp.maximum(
            pm_ref[g:g + 1, HF:F],
            jnp.max(jnp.where(mg, rr, -jnp.inf), axis=0, keepdims=True))
        pc_ref[g:g + 1, :] = pc_ref[g:g + 1, :] + jnp.sum(
            jnp.where(mg, 1.0, 0.0), axis=0, keepdims=True)


def _final_body(st_ref, ps_ref, pm_ref, pc_ref, g_ref, b_ref, wt_ref, lb_ref,
                cat_ref, out_ref):
    m2 = st_ref[0:1, :] / N
    v2 = st_ref[1:2, :] / N - m2 * m2
    sv = g_ref[...] * lax.rsqrt(v2 + EPS)
    tv = b_ref[...] - m2 * sv
    cnt = pc_ref[...]
    s_h = ps_ref[...] * sv + cnt * tv
    mx_h = pm_ref[...] * sv + tv
    mean_h = s_h / jnp.maximum(cnt, 1.0)
    cat_ref[:, 0:F] = s_h
    cat_ref[:, F:2 * F] = mean_h
    cat_ref[:, 2 * F:3 * F] = mx_h
    wt = wt_ref[...]
    out_ref[...] = (jnp.dot(s_h, wt[0:F], preferred_element_type=f32, precision=lax.Precision.HIGHEST)
                    + jnp.dot(mean_h, wt[F:2 * F], preferred_element_type=f32, precision=lax.Precision.HIGHEST)
                    + jnp.dot(mx_h, wt[2 * F:3 * F], preferred_element_type=f32, precision=lax.Precision.HIGHEST)
                    + lb_ref[...])


def _rowspec(width):
    return pl.BlockSpec((RB, width), lambda i: (i, 0))


def _fullspec(shape):
    return pl.BlockSpec(shape, lambda i: tuple(0 for _ in shape))


_SPLITSPEC = pl.BlockSpec((NC, RB, HF), lambda i: (0, i, 0))
_DEGSPEC = pl.BlockSpec((NC, RB, 16), lambda i: (0, i, 0))


def _step1(xp, B1, c1r, degp):
    return pl.pallas_call(
        _step1_body,
        grid=(GRID,),
        in_specs=[_rowspec(IN_F), _fullspec((IN_F, 192)), _fullspec((1, 192)),
                  _DEGSPEC],
        out_specs=[_SPLITSPEC, _rowspec(F), _rowspec(F)],
        out_shape=[jax.ShapeDtypeStruct((NC, NP, HF), f32),
                   jax.ShapeDtypeStruct((NP, F), f32),
                   jax.ShapeDtypeStruct((NP, F), f32)],
    )(xp, B1, c1r, degp)


def _step3(c0d, degp, sp):
    return pl.pallas_call(
        _step3_body,
        grid=(GRID,),
        in_specs=[_rowspec(F), _DEGSPEC, _SPLITSPEC],
        out_specs=[_SPLITSPEC],
        out_shape=[jax.ShapeDtypeStruct((NC, NP, HF), f32)],
    )(c0d, degp, sp)[0]


def _step5(d0, degp, sp):
    return pl.pallas_call(
        _step5_body,
        grid=(GRID,),
        in_specs=[_rowspec(F), _DEGSPEC, _SPLITSPEC],
        out_specs=[_rowspec(F), _fullspec((8, F))],
        out_shape=[jax.ShapeDtypeStruct((NP, F), f32),
                   jax.ShapeDtypeStruct((8, F), f32)],
    )(d0, degp, sp)


def _step6(r, B2, b2r, degp, st, g, b):
    return pl.pallas_call(
        _step6_body,
        grid=(GRID,),
        in_specs=[_rowspec(F), _fullspec((F, 192)), _fullspec((1, 192)),
                  _DEGSPEC, _fullspec((8, F)), _fullspec((1, F)),
                  _fullspec((1, F))],
        out_specs=[_SPLITSPEC, _rowspec(F), _rowspec(F)],
        out_shape=[jax.ShapeDtypeStruct((NC, NP, HF), f32),
                   jax.ShapeDtypeStruct((NP, F), f32),
                   jax.ShapeDtypeStruct((NP, F), f32)],
    )(r, B2, b2r, degp, st, g, b)


def _step10(d2o, degp, sp, batp):
    return pl.pallas_call(
        _step10_body,
        grid=(GRID,),
        in_specs=[_rowspec(F), _DEGSPEC, _SPLITSPEC, _rowspec(1)],
        out_specs=[_fullspec((8, F))] * 4,
        out_shape=[jax.ShapeDtypeStruct((8, F), f32)] * 4,
    )(d2o, degp, sp, batp)


def _final(st, ps, pm, pc, g, b, wt, lb):
    def fs(shape):
        return pl.BlockSpec(shape, lambda: tuple(0 for _ in shape))
    return pl.pallas_call(
        _final_body,
        in_specs=[
            fs((8, F)), fs((8, F)), fs((8, F)),
            fs((8, F)), fs((1, F)), fs((1, F)),
            fs((192, 32)), fs((1, 32))],
        out_specs=[fs((8, 192)), fs((8, 32))],
        out_shape=[jax.ShapeDtypeStruct((NG, 192), f32),
                   jax.ShapeDtypeStruct((NG, 32), f32)],
    )(st, ps, pm, pc, g, b, wt, lb)


@jax.jit
def kernel(x, edge_index, batch, sparse_mask, sm_weight, sm_bias,
           conv1_W, conv1_b, bn1_g, bn1_b, conv2_W, conv2_b, bn2_g, bn2_b,
           lin_W, lin_b):
    i32 = jnp.int32
    M = jnp.zeros((IN_F, IN_F), f32).at[sparse_mask[:, 0], sparse_mask[:, 1]].add(sm_weight)
    Wc1 = jnp.concatenate([2.0 * conv1_W[2], conv1_W[1], conv1_W[0] - conv1_W[2]], axis=1)
    B1 = jnp.dot(M, Wc1, precision=lax.Precision.HIGHEST)
    c1r = (jnp.dot(sm_bias, Wc1, precision=lax.Precision.HIGHEST) + jnp.concatenate(
        [jnp.zeros((F,), f32), jnp.zeros((F,), f32), conv1_b]))[None, :]
    B2 = jnp.concatenate([2.0 * conv2_W[2], conv2_W[1], conv2_W[0] - conv2_W[2]], axis=1)
    b2r = jnp.concatenate([jnp.zeros((F,), f32), jnp.zeros((F,), f32), conv2_b])[None, :]
    linWT = lin_W.T
    lbr = lin_b[None, :]
    g1 = bn1_g[None, :]; b1 = bn1_b[None, :]
    g2 = bn2_g[None, :]; b2 = bn2_b[None, :]

    xp = jnp.pad(x, ((0, NP - N), (0, 0)))
    batp = jnp.concatenate([batch, jnp.full((NP - N,), NG, i32)]).reshape(NP, 1)
    srcp = jnp.concatenate([edge_index[0], jnp.full((EP - E,), N, i32)])
    dstp = jnp.concatenate([edge_index[1], jnp.full((EP - E,), N, i32)])
    srci_d = srcp.reshape(NC * NS, EPT_D // 128, 128)
    src2i = jnp.stack([srcp, srcp + NP]).reshape(NC, NS, EPT // 128, 128)
    dsti = dstp.reshape(NS, EPT // 128, 128)
    z16 = jnp.zeros((RPS, 16), f32)
    z32 = jnp.zeros((RPS, HF), f32)
    ones16 = jnp.ones((128, 16), f32)

    degp = _deg_pass(ones16, z16, srci_d)
    ga, c0d, d0 = _step1(xp, B1, c1r, degp)
    sa = _scatter_pass(ga.reshape(NC * NP, HF), z32, src2i, dsti)
    gc = _step3(c0d, degp, sa)
    sc = _scatter_pass(gc.reshape(NC * NP, HF), z32, src2i, dsti)
    r, st1 = _step5(d0, degp, sc)
    ga2, c2d, d2o = _step6(r, B2, b2r, degp, st1, g1, b1)
    sa2 = _scatter_pass(ga2.reshape(NC * NP, HF), z32, src2i, dsti)
    gc2 = _step3(c2d, degp, sa2)
    sc2 = _scatter_pass(gc2.reshape(NC * NP, HF), z32, src2i, dsti)
    st2, ps, pm, pc = _step10(d2o, degp, sc2, batp)
    cat, out = _final(st2, ps, pm, pc, g2, b2, linWT, lbr)
    return cat, out


# speedup vs baseline: 10.6616x; 1.2677x over previous
import functools

import jax
import jax.numpy as jnp
from jax import lax
from jax.experimental import pallas as pl
from jax.experimental.pallas import tpu as pltpu
from jax.experimental.pallas import tpu_sc as plsc

N = 10000
E = 320000
IN_F = 128
F = 64
HF = 32
NG = 8
NP = 10240
EP = 327680
NC, NS = 2, 16
EPT = EP // NS
ROWS_IT = 8
CH = ROWS_IT * 128
N_IT = EPT // CH
RPS = NP // NS
EPT_D = EP // (NC * NS)
N_IT_D = EPT_D // CH
RB = 256
GRID = NP // RB
EPS = 1e-5
f32 = jnp.float32


def _sc_mesh():
    return plsc.VectorSubcoreMesh(core_axis_name="c", subcore_axis_name="s")


def _deg_body(ones_hbm, z_hbm, src_hbm, out_hbm, srcv, onesv, stage, acc, sem):
    c = lax.axis_index("c")
    s = lax.axis_index("s")
    w = c * NS + s
    pltpu.sync_copy(z_hbm, stage)
    pltpu.sync_copy(stage, acc.at[pl.ds(s * RPS, RPS)])
    pltpu.sync_copy(ones_hbm, onesv)
    plsc.subcore_barrier()

    @pl.loop(0, N_IT_D)
    def _(it):
        pltpu.sync_copy(src_hbm.at[w, pl.ds(it * ROWS_IT, ROWS_IT)], srcv)
        for j in range(ROWS_IT):
            pltpu.sync_copy(onesv, acc.at[srcv.at[j]], add=True)

    plsc.subcore_barrier()
    pltpu.sync_copy(acc.at[pl.ds(s * RPS, RPS)], stage)
    pltpu.sync_copy(stage, out_hbm.at[c, pl.ds(s * RPS, RPS)])


@jax.jit
def _deg_pass(ones16, z16, srci):
    kern = pl.kernel(
        _deg_body,
        out_type=jax.ShapeDtypeStruct((NC, NP, 16), f32),
        mesh=_sc_mesh(),
        compiler_params=pltpu.CompilerParams(use_tc_tiling_on_sc=False),
        scratch_types=[
            pltpu.VMEM((ROWS_IT, 128), jnp.int32),
            pltpu.VMEM((128, 16), f32),
            pltpu.VMEM((RPS, 16), f32),
            pltpu.VMEM_SHARED((NP, 16), f32),
            pltpu.SemaphoreType.DMA,
        ],
    )
    return kern(ones16, z16, srci)


def _scat_body(tab_hbm, z_hbm, src2_hbm, dst_hbm, out_hbm,
               srcv0, dstv0, srcv1, dstv1, rows0, rows1, stage, acc,
               gsem0, gsem1, ssem):
    c = lax.axis_index("c")
    s = lax.axis_index("s")
    srcv = (srcv0, srcv1)
    dstv = (dstv0, dstv1)
    rows = (rows0, rows1)
    gsem = (gsem0, gsem1)
    pltpu.sync_copy(z_hbm, stage)
    pltpu.sync_copy(stage, acc.at[pl.ds(s * RPS, RPS)])

    def load_idx(slot, itv):
        pltpu.sync_copy(src2_hbm.at[c, s, pl.ds(itv * ROWS_IT, ROWS_IT)],
                        srcv[slot])
        pltpu.sync_copy(dst_hbm.at[s, pl.ds(itv * ROWS_IT, ROWS_IT)],
                        dstv[slot])

    def fire_gathers(slot):
        for j in range(ROWS_IT):
            pltpu.async_copy(tab_hbm.at[srcv[slot].at[j]],
                             rows[slot].at[pl.ds(j * 128, 128)], gsem[slot])

    def wait_gathers(slot):
        for j in range(ROWS_IT):
            pltpu.make_async_copy(tab_hbm.at[srcv[slot].at[j]],
                                  rows[slot].at[pl.ds(j * 128, 128)],
                                  gsem[slot]).wait()

    plsc.subcore_barrier()
    load_idx(0, 0)
    fire_gathers(0)

    @pl.loop(0, N_IT, step=2)
    def _(it):
        for b in range(2):
            cur = it + b
            nb = 1 - b

            @pl.when(cur + 1 < N_IT)
            def _():
                load_idx(nb, cur + 1)
                fire_gathers(nb)

            wait_gathers(b)
            cps = []
            for j in range(ROWS_IT):
                cps.append(pltpu.async_copy(
                    rows[b].at[pl.ds(j * 128, 128)],
                    acc.at[dstv[b].at[j]], ssem, add=True))
            for cp in cps:
                cp.wait()

    plsc.subcore_barrier()
    pltpu.sync_copy(acc.at[pl.ds(s * RPS, RPS)], stage)
    pltpu.sync_copy(stage, out_hbm.at[c, pl.ds(s * RPS, RPS)])


@jax.jit
def _scatter_pass(tab, z32, src2i, dsti):
    kern = pl.kernel(
        _scat_body,
        out_type=jax.ShapeDtypeStruct((NC, NP, HF), f32),
        mesh=_sc_mesh(),
        compiler_params=pltpu.CompilerParams(use_tc_tiling_on_sc=False),
        scratch_types=[
            pltpu.VMEM((ROWS_IT, 128), jnp.int32),
            pltpu.VMEM((ROWS_IT, 128), jnp.int32),
            pltpu.VMEM((ROWS_IT, 128), jnp.int32),
            pltpu.VMEM((ROWS_IT, 128), jnp.int32),
            pltpu.VMEM((CH, HF), f32),
            pltpu.VMEM((CH, HF), f32),
            pltpu.VMEM((RPS, HF), f32),
            pltpu.VMEM_SHARED((NP, HF), f32),
            pltpu.SemaphoreType.DMA,
            pltpu.SemaphoreType.DMA,
            pltpu.SemaphoreType.DMA,
        ],
    )
    return kern(tab, z32, src2i, dsti)


def _dis(degp):
    deg = degp[0, :, 0:1] + degp[1, :, 0:1]
    return jnp.where(deg > 0, lax.rsqrt(jnp.maximum(deg, 1.0)), 0.0)


def _valid(i):
    row = lax.broadcasted_iota(jnp.int32, (RB, 1), 0) + i * RB
    return row < N


def _step1_body(x_ref, b1_ref, c1_ref, degp_ref, ga_ref, c0d_ref, d0_ref):
    i = pl.program_id(0)
    d = _dis(degp_ref[...])
    v = _valid(i)
    G = jnp.dot(x_ref[...], b1_ref[...], preferred_element_type=f32, precision=lax.Precision.HIGHEST) + c1_ref[...]
    ga_ref[0] = jnp.where(v, d * G[:, 0:HF], 0.0)
    ga_ref[1] = jnp.where(v, d * G[:, HF:F], 0.0)
    c0d_ref[...] = jnp.where(v, d * G[:, F:2 * F], 0.0)
    d0_ref[...] = jnp.where(v, G[:, 2 * F:3 * F], 0.0)


def _step3_body(c0d_ref, degp_ref, sp_ref, gc_ref):
    i = pl.program_id(0)
    d = _dis(degp_ref[...])
    v = _valid(i)
    d2 = d * d
    c0d = c0d_ref[...]
    gc_ref[0] = jnp.where(v, c0d[:, 0:HF] - d2 * sp_ref[0], 0.0)
    gc_ref[1] = jnp.where(v, c0d[:, HF:F] - d2 * sp_ref[1], 0.0)


def _step5_body(d0_ref, degp_ref, sp_ref, r_ref, st_ref):
    i = pl.program_id(0)
    d = _dis(degp_ref[...])
    v = _valid(i)
    d0 = d0_ref[...]
    rl = jnp.where(v, jnp.maximum(d0[:, 0:HF] - d * sp_ref[0], 0.0), 0.0)
    rr = jnp.where(v, jnp.maximum(d0[:, HF:F] - d * sp_ref[1], 0.0), 0.0)
    r_ref[:, 0:HF] = rl
    r_ref[:, HF:F] = rr

    @pl.when(i == 0)
    def _():
        st_ref[...] = jnp.zeros((8, F), f32)

    st_ref[0:1, 0:HF] = st_ref[0:1, 0:HF] + jnp.sum(rl, axis=0, keepdims=True)
    st_ref[0:1, HF:F] = st_ref[0:1, HF:F] + jnp.sum(rr, axis=0, keepdims=True)
    st_ref[1:2, 0:HF] = st_ref[1:2, 0:HF] + jnp.sum(rl * rl, axis=0, keepdims=True)
    st_ref[1:2, HF:F] = st_ref[1:2, HF:F] + jnp.sum(rr * rr, axis=0, keepdims=True)


def _step6_body(r_ref, b2_ref, b2r_ref, degp_ref, st_ref, g_ref, b_ref,
                ga2_ref, c2d_ref, d2o_ref):
    i = pl.program_id(0)
    d = _dis(degp_ref[...])
    v = _valid(i)
    m = st_ref[0:1, :] / N
    var = st_ref[1:2, :] / N - m * m
    sv = g_ref[...] * lax.rsqrt(var + EPS)
    tv = b_ref[...] - m * sv
    h1 = jnp.where(v, r_ref[...] * sv + tv, 0.0)
    G2 = jnp.dot(h1, b2_ref[...], preferred_element_type=f32, precision=lax.Precision.HIGHEST) + b2r_ref[...]
    ga2_ref[0] = jnp.where(v, d * G2[:, 0:HF], 0.0)
    ga2_ref[1] = jnp.where(v, d * G2[:, HF:F], 0.0)
    c2d_ref[...] = jnp.where(v, d * G2[:, F:2 * F], 0.0)
    d2o_ref[...] = jnp.where(v, G2[:, 2 * F:3 * F], 0.0)


def _step10_body(d2o_ref, degp_ref, sp_ref, bat_ref,
                 st_ref, ps_ref, pm_ref, pc_ref):
    i = pl.program_id(0)
    d = _dis(degp_ref[...])
    v = _valid(i)
    d2o = d2o_ref[...]
    rl = jnp.where(v, jnp.maximum(d2o[:, 0:HF] - d * sp_ref[0], 0.0), 0.0)
    rr = jnp.where(v, jnp.maximum(d2o[:, HF:F] - d * sp_ref[1], 0.0), 0.0)
    bat = bat_ref[...]

    @pl.when(i == 0)
    def _():
        st_ref[...] = jnp.zeros((8, F), f32)
        ps_ref[...] = jnp.zeros((8, F), f32)
        pm_ref[...] = jnp.full((8, F), -jnp.inf, f32)
        pc_ref[...] = jnp.zeros((8, F), f32)

    st_ref[0:1, 0:HF] = st_ref[0:1, 0:HF] + jnp.sum(rl, axis=0, keepdims=True)
    st_ref[0:1, HF:F] = st_ref[0:1, HF:F] + jnp.sum(rr, axis=0, keepdims=True)
    st_ref[1:2, 0:HF] = st_ref[1:2, 0:HF] + jnp.sum(rl * rl, axis=0, keepdims=True)
    st_ref[1:2, HF:F] = st_ref[1:2, HF:F] + jnp.sum(rr * rr, axis=0, keepdims=True)
    for g in range(NG):
        mg = bat == g
        ps_ref[g:g + 1, 0:HF] = ps_ref[g:g + 1, 0:HF] + jnp.sum(
            jnp.where(mg, rl, 0.0), axis=0, keepdims=True)
        ps_ref[g:g + 1, HF:F] = ps_ref[g:g + 1, HF:F] + jnp.sum(
            jnp.where(mg, rr, 0.0), axis=0, keepdims=True)
        pm_ref[g:g + 1, 0:HF] = jnp.maximum(
            pm_ref[g:g + 1, 0:HF],
            jnp.max(jnp.where(mg, rl, -jnp.inf), axis=0, keepdims=True))
        pm_ref[g:g + 1, HF:F] = jnp.maximum(
            pm_ref[g:g + 1, HF:F],
            jnp.max(jnp.where(mg, rr, -jnp.inf), axis=0, keepdims=True))
        pc_ref[g:g + 1, :] = pc_ref[g:g + 1, :] + jnp.sum(
            jnp.where(mg, 1.0, 0.0), axis=0, keepdims=True)


def _final_body(st_ref, ps_ref, pm_ref, pc_ref, g_ref, b_ref, wt_ref, lb_ref,
                cat_ref, out_ref):
    m2 = st_ref[0:1, :] / N
    v2 = st_ref[1:2, :] / N - m2 * m2
    sv = g_ref[...] * lax.rsqrt(v2 + EPS)
    tv = b_ref[...] - m2 * sv
    cnt = pc_ref[...]
    s_h = ps_ref[...] * sv + cnt * tv
    mx_h = pm_ref[...] * sv + tv
    mean_h = s_h / jnp.maximum(cnt, 1.0)
    cat_ref[:, 0:F] = s_h
    cat_ref[:, F:2 * F] = mean_h
    cat_ref[:, 2 * F:3 * F] = mx_h
    wt = wt_ref[...]
    out_ref[...] = (jnp.dot(s_h, wt[0:F], preferred_element_type=f32, precision=lax.Precision.HIGHEST)
                    + jnp.dot(mean_h, wt[F:2 * F], preferred_element_type=f32, precision=lax.Precision.HIGHEST)
                    + jnp.dot(mx_h, wt[2 * F:3 * F], preferred_element_type=f32, precision=lax.Precision.HIGHEST)
                    + lb_ref[...])


def _rowspec(width):
    return pl.BlockSpec((RB, width), lambda i: (i, 0))


def _fullspec(shape):
    return pl.BlockSpec(shape, lambda i: tuple(0 for _ in shape))


_SPLITSPEC = pl.BlockSpec((NC, RB, HF), lambda i: (0, i, 0))
_DEGSPEC = pl.BlockSpec((NC, RB, 16), lambda i: (0, i, 0))


def _step1(xp, B1, c1r, degp):
    return pl.pallas_call(
        _step1_body,
        grid=(GRID,),
        in_specs=[_rowspec(IN_F), _fullspec((IN_F, 192)), _fullspec((1, 192)),
                  _DEGSPEC],
        out_specs=[_SPLITSPEC, _rowspec(F), _rowspec(F)],
        out_shape=[jax.ShapeDtypeStruct((NC, NP, HF), f32),
                   jax.ShapeDtypeStruct((NP, F), f32),
                   jax.ShapeDtypeStruct((NP, F), f32)],
    )(xp, B1, c1r, degp)


def _step3(c0d, degp, sp):
    return pl.pallas_call(
        _step3_body,
        grid=(GRID,),
        in_specs=[_rowspec(F), _DEGSPEC, _SPLITSPEC],
        out_specs=[_SPLITSPEC],
        out_shape=[jax.ShapeDtypeStruct((NC, NP, HF), f32)],
    )(c0d, degp, sp)[0]


def _step5(d0, degp, sp):
    return pl.pallas_call(
        _step5_body,
        grid=(GRID,),
        in_specs=[_rowspec(F), _DEGSPEC, _SPLITSPEC],
        out_specs=[_rowspec(F), _fullspec((8, F))],
        out_shape=[jax.ShapeDtypeStruct((NP, F), f32),
                   jax.ShapeDtypeStruct((8, F), f32)],
    )(d0, degp, sp)


def _step6(r, B2, b2r, degp, st, g, b):
    return pl.pallas_call(
        _step6_body,
        grid=(GRID,),
        in_specs=[_rowspec(F), _fullspec((F, 192)), _fullspec((1, 192)),
                  _DEGSPEC, _fullspec((8, F)), _fullspec((1, F)),
                  _fullspec((1, F))],
        out_specs=[_SPLITSPEC, _rowspec(F), _rowspec(F)],
        out_shape=[jax.ShapeDtypeStruct((NC, NP, HF), f32),
                   jax.ShapeDtypeStruct((NP, F), f32),
                   jax.ShapeDtypeStruct((NP, F), f32)],
    )(r, B2, b2r, degp, st, g, b)


def _step10(d2o, degp, sp, batp):
    return pl.pallas_call(
        _step10_body,
        grid=(GRID,),
        in_specs=[_rowspec(F), _DEGSPEC, _SPLITSPEC, _rowspec(1)],
        out_specs=[_fullspec((8, F))] * 4,
        out_shape=[jax.ShapeDtypeStruct((8, F), f32)] * 4,
    )(d2o, degp, sp, batp)


def _final(st, ps, pm, pc, g, b, wt, lb):
    def fs(shape):
        return pl.BlockSpec(shape, lambda: tuple(0 for _ in shape))
    return pl.pallas_call(
        _final_body,
        in_specs=[
            fs((8, F)), fs((8, F)), fs((8, F)),
            fs((8, F)), fs((1, F)), fs((1, F)),
            fs((192, 32)), fs((1, 32))],
        out_specs=[fs((8, 192)), fs((8, 32))],
        out_shape=[jax.ShapeDtypeStruct((NG, 192), f32),
                   jax.ShapeDtypeStruct((NG, 32), f32)],
    )(st, ps, pm, pc, g, b, wt, lb)


@jax.jit
def kernel(x, edge_index, batch, sparse_mask, sm_weight, sm_bias,
           conv1_W, conv1_b, bn1_g, bn1_b, conv2_W, conv2_b, bn2_g, bn2_b,
           lin_W, lin_b):
    i32 = jnp.int32
    cols = jnp.arange(IN_F, dtype=jnp.int32)
    oh_in = (sparse_mask[:, 0:1] == cols[None, :]).astype(f32)
    oh_out = (sparse_mask[:, 1:2] == cols[None, :]).astype(f32) * sm_weight[:, None]
    M = jnp.dot(oh_in.T, oh_out, precision=lax.Precision.HIGHEST)
    Wc1 = jnp.concatenate([2.0 * conv1_W[2], conv1_W[1], conv1_W[0] - conv1_W[2]], axis=1)
    B1 = jnp.dot(M, Wc1, precision=lax.Precision.HIGHEST)
    c1r = (jnp.dot(sm_bias, Wc1, precision=lax.Precision.HIGHEST) + jnp.concatenate(
        [jnp.zeros((F,), f32), jnp.zeros((F,), f32), conv1_b]))[None, :]
    B2 = jnp.concatenate([2.0 * conv2_W[2], conv2_W[1], conv2_W[0] - conv2_W[2]], axis=1)
    b2r = jnp.concatenate([jnp.zeros((F,), f32), jnp.zeros((F,), f32), conv2_b])[None, :]
    linWT = lin_W.T
    lbr = lin_b[None, :]
    g1 = bn1_g[None, :]; b1 = bn1_b[None, :]
    g2 = bn2_g[None, :]; b2 = bn2_b[None, :]

    xp = jnp.pad(x, ((0, NP - N), (0, 0)))
    batp = jnp.concatenate([batch, jnp.full((NP - N,), NG, i32)]).reshape(NP, 1)
    srcp = jnp.concatenate([edge_index[0], jnp.full((EP - E,), N, i32)])
    dstp = jnp.concatenate([edge_index[1], jnp.full((EP - E,), N, i32)])
    srci_d = srcp.reshape(NC * NS, EPT_D // 128, 128)
    src2i = jnp.stack([srcp, srcp + NP]).reshape(NC, NS, EPT // 128, 128)
    dsti = dstp.reshape(NS, EPT // 128, 128)
    z16 = jnp.zeros((RPS, 16), f32)
    z32 = jnp.zeros((RPS, HF), f32)
    ones16 = jnp.ones((128, 16), f32)

    degp = _deg_pass(ones16, z16, srci_d)
    ga, c0d, d0 = _step1(xp, B1, c1r, degp)
    sa = _scatter_pass(ga.reshape(NC * NP, HF), z32, src2i, dsti)
    gc = _step3(c0d, degp, sa)
    sc = _scatter_pass(gc.reshape(NC * NP, HF), z32, src2i, dsti)
    r, st1 = _step5(d0, degp, sc)
    ga2, c2d, d2o = _step6(r, B2, b2r, degp, st1, g1, b1)
    sa2 = _scatter_pass(ga2.reshape(NC * NP, HF), z32, src2i, dsti)
    gc2 = _step3(c2d, degp, sa2)
    sc2 = _scatter_pass(gc2.reshape(NC * NP, HF), z32, src2i, dsti)
    st2, ps, pm, pc = _step10(d2o, degp, sc2, batp)
    cat, out = _final(st2, ps, pm, pc, g2, b2, linWT, lbr)
    return cat, out


# speedup vs baseline: 11.3154x; 1.0613x over previous
import functools

import jax
import jax.numpy as jnp
from jax import lax
from jax.experimental import pallas as pl
from jax.experimental.pallas import tpu as pltpu
from jax.experimental.pallas import tpu_sc as plsc

N = 10000
E = 320000
IN_F = 128
F = 64
HF = 32
NG = 8
NP = 10240
EP = 327680
NC, NS = 2, 16
EPT = EP // NS
ROWS_IT = 8
CH = ROWS_IT * 128
N_IT = EPT // CH
RPS = NP // NS
EPT_D = EP // (NC * NS)
N_IT_D = EPT_D // CH
RB = 256
GRID = NP // RB
EPS = 1e-5
f32 = jnp.float32


def _sc_mesh():
    return plsc.VectorSubcoreMesh(core_axis_name="c", subcore_axis_name="s")


def _deg_body(ones_hbm, z_hbm, src_hbm, out_hbm, srcv, onesv, stage, acc, sem):
    c = lax.axis_index("c")
    s = lax.axis_index("s")
    w = c * NS + s
    pltpu.sync_copy(z_hbm, stage)
    pltpu.sync_copy(stage, acc.at[pl.ds(s * RPS, RPS)])
    pltpu.sync_copy(ones_hbm, onesv)
    plsc.subcore_barrier()

    @pl.loop(0, N_IT_D)
    def _(it):
        pltpu.sync_copy(src_hbm.at[w, pl.ds(it * ROWS_IT, ROWS_IT)], srcv)
        for j in range(ROWS_IT):
            pltpu.sync_copy(onesv, acc.at[srcv.at[j]], add=True)

    plsc.subcore_barrier()
    pltpu.sync_copy(acc.at[pl.ds(s * RPS, RPS)], stage)
    pltpu.sync_copy(stage, out_hbm.at[c, pl.ds(s * RPS, RPS)])


@jax.jit
def _deg_pass(ones16, z16, srci):
    kern = pl.kernel(
        _deg_body,
        out_type=jax.ShapeDtypeStruct((NC, NP, 16), f32),
        mesh=_sc_mesh(),
        compiler_params=pltpu.CompilerParams(use_tc_tiling_on_sc=False),
        scratch_types=[
            pltpu.VMEM((ROWS_IT, 128), jnp.int32),
            pltpu.VMEM((128, 16), f32),
            pltpu.VMEM((RPS, 16), f32),
            pltpu.VMEM_SHARED((NP, 16), f32),
            pltpu.SemaphoreType.DMA,
        ],
    )
    return kern(ones16, z16, srci)


HS = RPS // 4


def _conv_body(tab_hbm, cm_hbm, d2b_hbm, src2_hbm, dst_hbm,
               out_hbm, gc_hbm,
               srcv0, dstv0, srcv1, dstv1, rows0, rows1, stage,
               accv, cmv, d2v, acc, gsem0, gsem1, ssem):
    c = lax.axis_index("c")
    s = lax.axis_index("s")
    srcv = (srcv0, srcv1)
    dstv = (dstv0, dstv1)
    rows = (rows0, rows1)
    gsem = (gsem0, gsem1)
    zv = jnp.zeros((16,), f32)

    @pl.loop(0, RPS)
    def _(rr):
        stage[rr, pl.ds(0, 16)] = zv
        stage[rr, pl.ds(16, 16)] = zv

    pltpu.sync_copy(stage, acc.at[pl.ds(s * RPS, RPS)])

    def scatter_sweep(t_ref, srcarr):
        def load_idx(slot, itv):
            pltpu.sync_copy(srcarr.at[c, s, pl.ds(itv * ROWS_IT, ROWS_IT)],
                            srcv[slot])
            pltpu.sync_copy(dst_hbm.at[c, s, pl.ds(itv * ROWS_IT, ROWS_IT)],
                            dstv[slot])

        def fire_gathers(slot):
            for j in range(ROWS_IT):
                pltpu.async_copy(t_ref.at[srcv[slot].at[j]],
                                 rows[slot].at[pl.ds(j * 128, 128)],
                                 gsem[slot])

        def wait_gathers(slot):
            for j in range(ROWS_IT):
                pltpu.make_async_copy(t_ref.at[srcv[slot].at[j]],
                                      rows[slot].at[pl.ds(j * 128, 128)],
                                      gsem[slot]).wait()

        load_idx(0, 0)
        fire_gathers(0)

        @pl.loop(0, N_IT, step=2)
        def _(it):
            for b in range(2):
                cur = it + b
                nb = 1 - b

                @pl.when(cur + 1 < N_IT)
                def _():
                    load_idx(nb, cur + 1)
                    fire_gathers(nb)

                wait_gathers(b)
                cps = []
                for j in range(ROWS_IT):
                    cps.append(pltpu.async_copy(
                        rows[b].at[pl.ds(j * 128, 128)],
                        acc.at[dstv[b].at[j]], ssem, add=True))
                for cp in cps:
                    cp.wait()

    plsc.subcore_barrier()
    scatter_sweep(tab_hbm, src2_hbm)
    plsc.subcore_barrier()

    for half in range(4):
        base = s * RPS + half * HS
        pltpu.sync_copy(acc.at[pl.ds(base, HS)], accv)
        pltpu.sync_copy(cm_hbm.at[pl.ds(c * NP + base, HS)], cmv)
        pltpu.sync_copy(d2b_hbm.at[pl.ds(base, HS)], d2v)

        @pl.loop(0, HS)
        def _(rr):
            for k in range(2):
                sl = pl.ds(k * 16, 16)
                cmv[rr, sl] = cmv[rr, sl] - d2v[rr, sl] * accv[rr, sl]

        pltpu.sync_copy(cmv, gc_hbm.at[pl.ds(c * NP + base, HS)])

    pltpu.sync_copy(stage, acc.at[pl.ds(s * RPS, RPS)])
    plsc.subcore_barrier()
    scatter_sweep(gc_hbm, src2_hbm)
    plsc.subcore_barrier()
    pltpu.sync_copy(acc.at[pl.ds(s * RPS, RPS)], stage)
    pltpu.sync_copy(stage, out_hbm.at[c, pl.ds(s * RPS, RPS)])


@jax.jit
def _conv_pass(tab, cm, d2b, src2i, dsti):
    kern = pl.kernel(
        _conv_body,
        out_type=[jax.ShapeDtypeStruct((NC, NP, HF), f32),
                  jax.ShapeDtypeStruct((NC * NP, HF), f32)],
        mesh=_sc_mesh(),
        compiler_params=pltpu.CompilerParams(use_tc_tiling_on_sc=False),
        scratch_types=[
            pltpu.VMEM((ROWS_IT, 128), jnp.int32),
            pltpu.VMEM((ROWS_IT, 128), jnp.int32),
            pltpu.VMEM((ROWS_IT, 128), jnp.int32),
            pltpu.VMEM((ROWS_IT, 128), jnp.int32),
            pltpu.VMEM((CH, HF), f32),
            pltpu.VMEM((CH, HF), f32),
            pltpu.VMEM((RPS, HF), f32),
            pltpu.VMEM((HS, HF), f32),
            pltpu.VMEM((HS, HF), f32),
            pltpu.VMEM((HS, HF), f32),
            pltpu.VMEM_SHARED((NP, HF), f32),
            pltpu.SemaphoreType.DMA,
            pltpu.SemaphoreType.DMA,
            pltpu.SemaphoreType.DMA,
        ],
    )
    return kern(tab, cm, d2b, src2i, dsti)


def _dis(degp):
    deg = degp[0, :, 0:1] + degp[1, :, 0:1]
    return jnp.where(deg > 0, lax.rsqrt(jnp.maximum(deg, 1.0)), 0.0)


def _valid(i):
    row = lax.broadcasted_iota(jnp.int32, (RB, 1), 0) + i * RB
    return row < N


def _step1_body(x_ref, b1_ref, c1_ref, degp_ref, ga_ref, cm_ref, d2b_ref,
                d0_ref):
    i = pl.program_id(0)
    d = _dis(degp_ref[...])
    v = _valid(i)
    G = jnp.dot(x_ref[...], b1_ref[...], preferred_element_type=f32, precision=lax.Precision.HIGHEST) + c1_ref[...]
    ga_ref[0] = jnp.where(v, d * G[:, 0:HF], 0.0)
    ga_ref[1] = jnp.where(v, d * G[:, HF:F], 0.0)
    cm_ref[0] = jnp.where(v, d * G[:, F:F + HF], 0.0)
    cm_ref[1] = jnp.where(v, d * G[:, F + HF:2 * F], 0.0)
    d2b_ref[...] = jnp.broadcast_to(d * d, (RB, HF))
    d0_ref[...] = jnp.where(v, G[:, 2 * F:3 * F], 0.0)


def _step5_body(d0_ref, degp_ref, sp_ref, r_ref, st_ref):
    i = pl.program_id(0)
    d = _dis(degp_ref[...])
    v = _valid(i)
    d0 = d0_ref[...]
    rl = jnp.where(v, jnp.maximum(d0[:, 0:HF] - d * sp_ref[0], 0.0), 0.0)
    rr = jnp.where(v, jnp.maximum(d0[:, HF:F] - d * sp_ref[1], 0.0), 0.0)
    r_ref[:, 0:HF] = rl
    r_ref[:, HF:F] = rr

    @pl.when(i == 0)
    def _():
        st_ref[...] = jnp.zeros((8, F), f32)

    st_ref[0:1, 0:HF] = st_ref[0:1, 0:HF] + jnp.sum(rl, axis=0, keepdims=True)
    st_ref[0:1, HF:F] = st_ref[0:1, HF:F] + jnp.sum(rr, axis=0, keepdims=True)
    st_ref[1:2, 0:HF] = st_ref[1:2, 0:HF] + jnp.sum(rl * rl, axis=0, keepdims=True)
    st_ref[1:2, HF:F] = st_ref[1:2, HF:F] + jnp.sum(rr * rr, axis=0, keepdims=True)


def _step6_body(r_ref, b2_ref, b2r_ref, degp_ref, st_ref, g_ref, b_ref,
                ga2_ref, c2d_ref, d2o_ref):
    i = pl.program_id(0)
    d = _dis(degp_ref[...])
    v = _valid(i)
    m = st_ref[0:1, :] / N
    var = st_ref[1:2, :] / N - m * m
    sv = g_ref[...] * lax.rsqrt(var + EPS)
    tv = b_ref[...] - m * sv
    h1 = jnp.where(v, r_ref[...] * sv + tv, 0.0)
    G2 = jnp.dot(h1, b2_ref[...], preferred_element_type=f32, precision=lax.Precision.HIGHEST) + b2r_ref[...]
    ga2_ref[0] = jnp.where(v, d * G2[:, 0:HF], 0.0)
    ga2_ref[1] = jnp.where(v, d * G2[:, HF:F], 0.0)
    c2d_ref[0] = jnp.where(v, d * G2[:, F:F + HF], 0.0)
    c2d_ref[1] = jnp.where(v, d * G2[:, F + HF:2 * F], 0.0)
    d2o_ref[...] = jnp.where(v, G2[:, 2 * F:3 * F], 0.0)


def _step10_body(d2o_ref, degp_ref, sp_ref, bat_ref,
                 st_ref, ps_ref, pm_ref, pc_ref):
    i = pl.program_id(0)
    d = _dis(degp_ref[...])
    v = _valid(i)
    d2o = d2o_ref[...]
    rl = jnp.where(v, jnp.maximum(d2o[:, 0:HF] - d * sp_ref[0], 0.0), 0.0)
    rr = jnp.where(v, jnp.maximum(d2o[:, HF:F] - d * sp_ref[1], 0.0), 0.0)
    bat = bat_ref[...]

    @pl.when(i == 0)
    def _():
        st_ref[...] = jnp.zeros((8, F), f32)
        ps_ref[...] = jnp.zeros((8, F), f32)
        pm_ref[...] = jnp.full((8, F), -jnp.inf, f32)
        pc_ref[...] = jnp.zeros((8, F), f32)

    st_ref[0:1, 0:HF] = st_ref[0:1, 0:HF] + jnp.sum(rl, axis=0, keepdims=True)
    st_ref[0:1, HF:F] = st_ref[0:1, HF:F] + jnp.sum(rr, axis=0, keepdims=True)
    st_ref[1:2, 0:HF] = st_ref[1:2, 0:HF] + jnp.sum(rl * rl, axis=0, keepdims=True)
    st_ref[1:2, HF:F] = st_ref[1:2, HF:F] + jnp.sum(rr * rr, axis=0, keepdims=True)
    for g in range(NG):
        mg = bat == g
        ps_ref[g:g + 1, 0:HF] = ps_ref[g:g + 1, 0:HF] + jnp.sum(
            jnp.where(mg, rl, 0.0), axis=0, keepdims=True)
        ps_ref[g:g + 1, HF:F] = ps_ref[g:g + 1, HF:F] + jnp.sum(
            jnp.where(mg, rr, 0.0), axis=0, keepdims=True)
        pm_ref[g:g + 1, 0:HF] = jnp.maximum(
            pm_ref[g:g + 1, 0:HF],
            jnp.max(jnp.where(mg, rl, -jnp.inf), axis=0, keepdims=True))
        pm_ref[g:g + 1, HF:F] = jnp.maximum(
            pm_ref[g:g + 1, HF:F],
            jnp.max(jnp.where(mg, rr, -jnp.inf), axis=0, keepdims=True))
        pc_ref[g:g + 1, :] = pc_ref[g:g + 1, :] + jnp.sum(
            jnp.where(mg, 1.0, 0.0), axis=0, keepdims=True)


def _final_body(st_ref, ps_ref, pm_ref, pc_ref, g_ref, b_ref, wt_ref, lb_ref,
                cat_ref, out_ref):
    m2 = st_ref[0:1, :] / N
    v2 = st_ref[1:2, :] / N - m2 * m2
    sv = g_ref[...] * lax.rsqrt(v2 + EPS)
    tv = b_ref[...] - m2 * sv
    cnt = pc_ref[...]
    s_h = ps_ref[...] * sv + cnt * tv
    mx_h = pm_ref[...] * sv + tv
    mean_h = s_h / jnp.maximum(cnt, 1.0)
    cat_ref[:, 0:F] = s_h
    cat_ref[:, F:2 * F] = mean_h
    cat_ref[:, 2 * F:3 * F] = mx_h
    wt = wt_ref[...]
    out_ref[...] = (jnp.dot(s_h, wt[0:F], preferred_element_type=f32, precision=lax.Precision.HIGHEST)
                    + jnp.dot(mean_h, wt[F:2 * F], preferred_element_type=f32, precision=lax.Precision.HIGHEST)
                    + jnp.dot(mx_h, wt[2 * F:3 * F], preferred_element_type=f32, precision=lax.Precision.HIGHEST)
                    + lb_ref[...])


def _rowspec(width):
    return pl.BlockSpec((RB, width), lambda i: (i, 0))


def _fullspec(shape):
    return pl.BlockSpec(shape, lambda i: tuple(0 for _ in shape))


_SPLITSPEC = pl.BlockSpec((NC, RB, HF), lambda i: (0, i, 0))
_DEGSPEC = pl.BlockSpec((NC, RB, 16), lambda i: (0, i, 0))


def _step1(xp, B1, c1r, degp):
    return pl.pallas_call(
        _step1_body,
        grid=(GRID,),
        in_specs=[_rowspec(IN_F), _fullspec((IN_F, 192)), _fullspec((1, 192)),
                  _DEGSPEC],
        out_specs=[_SPLITSPEC, _SPLITSPEC, _rowspec(HF), _rowspec(F)],
        out_shape=[jax.ShapeDtypeStruct((NC, NP, HF), f32),
                   jax.ShapeDtypeStruct((NC, NP, HF), f32),
                   jax.ShapeDtypeStruct((NP, HF), f32),
                   jax.ShapeDtypeStruct((NP, F), f32)],
    )(xp, B1, c1r, degp)


def _step5(d0, degp, sp):
    return pl.pallas_call(
        _step5_body,
        grid=(GRID,),
        in_specs=[_rowspec(F), _DEGSPEC, _SPLITSPEC],
        out_specs=[_rowspec(F), _fullspec((8, F))],
        out_shape=[jax.ShapeDtypeStruct((NP, F), f32),
                   jax.ShapeDtypeStruct((8, F), f32)],
    )(d0, degp, sp)


def _step6(r, B2, b2r, degp, st, g, b):
    return pl.pallas_call(
        _step6_body,
        grid=(GRID,),
        in_specs=[_rowspec(F), _fullspec((F, 192)), _fullspec((1, 192)),
                  _DEGSPEC, _fullspec((8, F)), _fullspec((1, F)),
                  _fullspec((1, F))],
        out_specs=[_SPLITSPEC, _SPLITSPEC, _rowspec(F)],
        out_shape=[jax.ShapeDtypeStruct((NC, NP, HF), f32),
                   jax.ShapeDtypeStruct((NC, NP, HF), f32),
                   jax.ShapeDtypeStruct((NP, F), f32)],
    )(r, B2, b2r, degp, st, g, b)


def _step10(d2o, degp, sp, batp):
    return pl.pallas_call(
        _step10_body,
        grid=(GRID,),
        in_specs=[_rowspec(F), _DEGSPEC, _SPLITSPEC, _rowspec(1)],
        out_specs=[_fullspec((8, F))] * 4,
        out_shape=[jax.ShapeDtypeStruct((8, F), f32)] * 4,
    )(d2o, degp, sp, batp)


def _final(st, ps, pm, pc, g, b, wt, lb):
    def fs(shape):
        return pl.BlockSpec(shape, lambda: tuple(0 for _ in shape))
    return pl.pallas_call(
        _final_body,
        in_specs=[
            fs((8, F)), fs((8, F)), fs((8, F)),
            fs((8, F)), fs((1, F)), fs((1, F)),
            fs((192, 32)), fs((1, 32))],
        out_specs=[fs((8, 192)), fs((8, 32))],
        out_shape=[jax.ShapeDtypeStruct((NG, 192), f32),
                   jax.ShapeDtypeStruct((NG, 32), f32)],
    )(st, ps, pm, pc, g, b, wt, lb)


@jax.jit
def kernel(x, edge_index, batch, sparse_mask, sm_weight, sm_bias,
           conv1_W, conv1_b, bn1_g, bn1_b, conv2_W, conv2_b, bn2_g, bn2_b,
           lin_W, lin_b):
    i32 = jnp.int32
    cols = jnp.arange(IN_F, dtype=jnp.int32)
    oh_in = (sparse_mask[:, 0:1] == cols[None, :]).astype(f32)
    oh_out = (sparse_mask[:, 1:2] == cols[None, :]).astype(f32) * sm_weight[:, None]
    M = jnp.dot(oh_in.T, oh_out, precision=lax.Precision.HIGHEST)
    Wc1 = jnp.concatenate([2.0 * conv1_W[2], conv1_W[1], conv1_W[0] - conv1_W[2]], axis=1)
    B1 = jnp.dot(M, Wc1, precision=lax.Precision.HIGHEST)
    c1r = (jnp.dot(sm_bias, Wc1, precision=lax.Precision.HIGHEST) + jnp.concatenate(
        [jnp.zeros((F,), f32), jnp.zeros((F,), f32), conv1_b]))[None, :]
    B2 = jnp.concatenate([2.0 * conv2_W[2], conv2_W[1], conv2_W[0] - conv2_W[2]], axis=1)
    b2r = jnp.concatenate([jnp.zeros((F,), f32), jnp.zeros((F,), f32), conv2_b])[None, :]
    linWT = lin_W.T
    lbr = lin_b[None, :]
    g1 = bn1_g[None, :]; b1 = bn1_b[None, :]
    g2 = bn2_g[None, :]; b2 = bn2_b[None, :]

    xp = jnp.pad(x, ((0, NP - N), (0, 0)))
    batp = jnp.concatenate([batch, jnp.full((NP - N,), NG, i32)]).reshape(NP, 1)
    srcp = jnp.concatenate([edge_index[0], jnp.full((EP - E,), N, i32)])
    dstp = jnp.concatenate([edge_index[1], jnp.full((EP - E,), N, i32)])
    srci_d = srcp.reshape(NC * NS, EPT_D // 128, 128)
    src2i = jnp.stack([srcp, srcp + NP]).reshape(NC, NS, EPT // 128, 128)
    dsti = jnp.broadcast_to(dstp.reshape(1, NS, EPT // 128, 128),
                            (NC, NS, EPT // 128, 128))
    z16 = jnp.zeros((RPS, 16), f32)
    ones16 = jnp.ones((128, 16), f32)

    degp = _deg_pass(ones16, z16, srci_d)
    ga, cm1, d2b, d0 = _step1(xp, B1, c1r, degp)
    sc, _ = _conv_pass(ga.reshape(NC * NP, HF),
                       cm1.reshape(NC * NP, HF), d2b, src2i, dsti)
    r, st1 = _step5(d0, degp, sc)
    ga2, cm2, d2o = _step6(r, B2, b2r, degp, st1, g1, b1)
    sc2, _ = _conv_pass(ga2.reshape(NC * NP, HF),
                        cm2.reshape(NC * NP, HF), d2b, src2i, dsti)
    st2, ps, pm, pc = _step10(d2o, degp, sc2, batp)
    cat, out = _final(st2, ps, pm, pc, g2, b2, linWT, lbr)
    return cat, out


# speedup vs baseline: 11.9743x; 1.0582x over previous
import functools

import jax
import jax.numpy as jnp
from jax import lax
from jax.experimental import pallas as pl
from jax.experimental.pallas import tpu as pltpu
from jax.experimental.pallas import tpu_sc as plsc

N = 10000
E = 320000
IN_F = 128
F = 64
HF = 32
NG = 8
NP = 10240
EP = 327680
NC, NS = 2, 16
EPT = EP // NS
ROWS_IT = 8
CH = ROWS_IT * 128
N_IT = EPT // CH
RPS = NP // NS
EPT_D = EP // (NC * NS)
N_IT_D = EPT_D // CH
RB = 256
GRID = NP // RB
EPS = 1e-5
f32 = jnp.float32


def _sc_mesh():
    return plsc.VectorSubcoreMesh(core_axis_name="c", subcore_axis_name="s")


def _deg_body(ones_hbm, z_hbm, src_hbm, out_hbm, srcv, onesv, stage, acc, sem):
    c = lax.axis_index("c")
    s = lax.axis_index("s")
    w = c * NS + s
    pltpu.sync_copy(z_hbm, stage)
    pltpu.sync_copy(stage, acc.at[pl.ds(s * RPS, RPS)])
    pltpu.sync_copy(ones_hbm, onesv)
    plsc.subcore_barrier()

    @pl.loop(0, N_IT_D)
    def _(it):
        pltpu.sync_copy(src_hbm.at[w, pl.ds(it * ROWS_IT, ROWS_IT)], srcv)
        for j in range(ROWS_IT):
            pltpu.sync_copy(onesv, acc.at[srcv.at[j]], add=True)

    plsc.subcore_barrier()
    pltpu.sync_copy(acc.at[pl.ds(s * RPS, RPS)], stage)
    pltpu.sync_copy(stage, out_hbm.at[c, pl.ds(s * RPS, RPS)])


@jax.jit
def _deg_pass(ones16, z16, srci):
    kern = pl.kernel(
        _deg_body,
        out_type=jax.ShapeDtypeStruct((NC, NP, 16), f32),
        mesh=_sc_mesh(),
        compiler_params=pltpu.CompilerParams(use_tc_tiling_on_sc=False),
        scratch_types=[
            pltpu.VMEM((ROWS_IT, 128), jnp.int32),
            pltpu.VMEM((128, 16), f32),
            pltpu.VMEM((RPS, 16), f32),
            pltpu.VMEM_SHARED((NP, 16), f32),
            pltpu.SemaphoreType.DMA,
        ],
    )
    return kern(ones16, z16, srci)


HS = RPS // 4


def _conv_body(tab_hbm, cm_hbm, d2b_hbm, src_hbm, dst_hbm,
               out_hbm, gc_hbm,
               srcv0, dstv0, srcv1, dstv1, rows0, rows1, stage,
               accv, cmv, d2v, acc, gsem0, gsem1, ssem):
    c = lax.axis_index("c")
    s = lax.axis_index("s")
    srcv = (srcv0, srcv1)
    dstv = (dstv0, dstv1)
    rows = (rows0, rows1)
    gsem = (gsem0, gsem1)
    zv = jnp.zeros((16,), f32)

    @pl.loop(0, RPS)
    def _(rr):
        stage[rr, pl.ds(0, 16)] = zv
        stage[rr, pl.ds(16, 16)] = zv

    pltpu.sync_copy(stage, acc.at[pl.ds(s * RPS, RPS)])

    def scatter_sweep(t_ref):
        def load_idx(slot, itv):
            pltpu.sync_copy(src_hbm.at[s, pl.ds(itv * ROWS_IT, ROWS_IT)],
                            srcv[slot])
            pltpu.sync_copy(dst_hbm.at[s, pl.ds(itv * ROWS_IT, ROWS_IT)],
                            dstv[slot])

        def fire_gathers(slot):
            for j in range(ROWS_IT):
                pltpu.async_copy(t_ref.at[srcv[slot].at[j]],
                                 rows[slot].at[pl.ds(j * 128, 128)],
                                 gsem[slot])

        def wait_gathers(slot):
            for j in range(ROWS_IT):
                pltpu.make_async_copy(t_ref.at[srcv[slot].at[j]],
                                      rows[slot].at[pl.ds(j * 128, 128)],
                                      gsem[slot]).wait()

        load_idx(0, 0)
        fire_gathers(0)

        @pl.loop(0, N_IT, step=2)
        def _(it):
            for b in range(2):
                cur = it + b
                nb = 1 - b

                @pl.when(cur + 1 < N_IT)
                def _():
                    load_idx(nb, cur + 1)
                    fire_gathers(nb)

                wait_gathers(b)
                cps = []
                for j in range(ROWS_IT):
                    cps.append(pltpu.async_copy(
                        rows[b].at[pl.ds(j * 128, 128)],
                        acc.at[dstv[b].at[j]], ssem, add=True))
                for cp in cps:
                    cp.wait()

    plsc.subcore_barrier()
    scatter_sweep(tab_hbm.at[c])
    plsc.subcore_barrier()

    for half in range(4):
        base = s * RPS + half * HS
        pltpu.sync_copy(acc.at[pl.ds(base, HS)], accv)
        pltpu.sync_copy(cm_hbm.at[c, pl.ds(base, HS)], cmv)
        pltpu.sync_copy(d2b_hbm.at[pl.ds(base, HS)], d2v)

        @pl.loop(0, HS)
        def _(rr):
            for k in range(2):
                sl = pl.ds(k * 16, 16)
                cmv[rr, sl] = cmv[rr, sl] - d2v[rr, sl] * accv[rr, sl]

        pltpu.sync_copy(cmv, gc_hbm.at[c, pl.ds(base, HS)])

    pltpu.sync_copy(stage, acc.at[pl.ds(s * RPS, RPS)])
    plsc.subcore_barrier()
    scatter_sweep(gc_hbm.at[c])
    plsc.subcore_barrier()
    pltpu.sync_copy(acc.at[pl.ds(s * RPS, RPS)], stage)
    pltpu.sync_copy(stage, out_hbm.at[c, pl.ds(s * RPS, RPS)])


@jax.jit
def _conv_pass(tab, cm, d2b, srci, dsti):
    kern = pl.kernel(
        _conv_body,
        out_type=[jax.ShapeDtypeStruct((NC, NP, HF), f32),
                  jax.ShapeDtypeStruct((NC, NP, HF), f32)],
        mesh=_sc_mesh(),
        compiler_params=pltpu.CompilerParams(use_tc_tiling_on_sc=False),
        scratch_types=[
            pltpu.VMEM((ROWS_IT, 128), jnp.int32),
            pltpu.VMEM((ROWS_IT, 128), jnp.int32),
            pltpu.VMEM((ROWS_IT, 128), jnp.int32),
            pltpu.VMEM((ROWS_IT, 128), jnp.int32),
            pltpu.VMEM((CH, HF), f32),
            pltpu.VMEM((CH, HF), f32),
            pltpu.VMEM((RPS, HF), f32),
            pltpu.VMEM((HS, HF), f32),
            pltpu.VMEM((HS, HF), f32),
            pltpu.VMEM((HS, HF), f32),
            pltpu.VMEM_SHARED((NP, HF), f32),
            pltpu.SemaphoreType.DMA,
            pltpu.SemaphoreType.DMA,
            pltpu.SemaphoreType.DMA,
        ],
    )
    return kern(tab, cm, d2b, srci, dsti)


def _dis(degp):
    deg = degp[0, :, 0:1] + degp[1, :, 0:1]
    return jnp.where(deg > 0, lax.rsqrt(jnp.maximum(deg, 1.0)), 0.0)


def _valid(i):
    row = lax.broadcasted_iota(jnp.int32, (RB, 1), 0) + i * RB
    return row < N


def _step1_body(x_ref, b1_ref, c1_ref, degp_ref, ga_ref, cm_ref, d2b_ref,
                d0_ref):
    i = pl.program_id(0)
    d = _dis(degp_ref[...])
    v = _valid(i)
    G = jnp.dot(x_ref[...], b1_ref[...], preferred_element_type=f32, precision=lax.Precision.HIGHEST) + c1_ref[...]
    ga_ref[0] = jnp.where(v, d * G[:, 0:HF], 0.0)
    ga_ref[1] = jnp.where(v, d * G[:, HF:F], 0.0)
    cm_ref[0] = jnp.where(v, d * G[:, F:F + HF], 0.0)
    cm_ref[1] = jnp.where(v, d * G[:, F + HF:2 * F], 0.0)
    d2b_ref[...] = jnp.broadcast_to(d * d, (RB, HF))
    d0_ref[...] = jnp.where(v, G[:, 2 * F:3 * F], 0.0)


def _step5_body(d0_ref, degp_ref, sp_ref, r_ref, st_ref):
    i = pl.program_id(0)
    d = _dis(degp_ref[...])
    v = _valid(i)
    d0 = d0_ref[...]
    rl = jnp.where(v, jnp.maximum(d0[:, 0:HF] - d * sp_ref[0], 0.0), 0.0)
    rr = jnp.where(v, jnp.maximum(d0[:, HF:F] - d * sp_ref[1], 0.0), 0.0)
    r_ref[:, 0:HF] = rl
    r_ref[:, HF:F] = rr

    @pl.when(i == 0)
    def _():
        st_ref[...] = jnp.zeros((8, F), f32)

    st_ref[0:1, 0:HF] = st_ref[0:1, 0:HF] + jnp.sum(rl, axis=0, keepdims=True)
    st_ref[0:1, HF:F] = st_ref[0:1, HF:F] + jnp.sum(rr, axis=0, keepdims=True)
    st_ref[1:2, 0:HF] = st_ref[1:2, 0:HF] + jnp.sum(rl * rl, axis=0, keepdims=True)
    st_ref[1:2, HF:F] = st_ref[1:2, HF:F] + jnp.sum(rr * rr, axis=0, keepdims=True)


def _step6_body(r_ref, b2_ref, b2r_ref, degp_ref, st_ref, g_ref, b_ref,
                ga2_ref, c2d_ref, d2o_ref):
    i = pl.program_id(0)
    d = _dis(degp_ref[...])
    v = _valid(i)
    m = st_ref[0:1, :] / N
    var = st_ref[1:2, :] / N - m * m
    sv = g_ref[...] * lax.rsqrt(var + EPS)
    tv = b_ref[...] - m * sv
    h1 = jnp.where(v, r_ref[...] * sv + tv, 0.0)
    G2 = jnp.dot(h1, b2_ref[...], preferred_element_type=f32, precision=lax.Precision.HIGHEST) + b2r_ref[...]
    ga2_ref[0] = jnp.where(v, d * G2[:, 0:HF], 0.0)
    ga2_ref[1] = jnp.where(v, d * G2[:, HF:F], 0.0)
    c2d_ref[0] = jnp.where(v, d * G2[:, F:F + HF], 0.0)
    c2d_ref[1] = jnp.where(v, d * G2[:, F + HF:2 * F], 0.0)
    d2o_ref[...] = jnp.where(v, G2[:, 2 * F:3 * F], 0.0)


def _step10_body(d2o_ref, degp_ref, sp_ref, bat_ref,
                 st_ref, ps_ref, pm_ref, pc_ref):
    i = pl.program_id(0)
    d = _dis(degp_ref[...])
    v = _valid(i)
    d2o = d2o_ref[...]
    rl = jnp.where(v, jnp.maximum(d2o[:, 0:HF] - d * sp_ref[0], 0.0), 0.0)
    rr = jnp.where(v, jnp.maximum(d2o[:, HF:F] - d * sp_ref[1], 0.0), 0.0)
    bat = bat_ref[...]

    @pl.when(i == 0)
    def _():
        st_ref[...] = jnp.zeros((8, F), f32)
        ps_ref[...] = jnp.zeros((8, F), f32)
        pm_ref[...] = jnp.full((8, F), -jnp.inf, f32)
        pc_ref[...] = jnp.zeros((8, F), f32)

    st_ref[0:1, 0:HF] = st_ref[0:1, 0:HF] + jnp.sum(rl, axis=0, keepdims=True)
    st_ref[0:1, HF:F] = st_ref[0:1, HF:F] + jnp.sum(rr, axis=0, keepdims=True)
    st_ref[1:2, 0:HF] = st_ref[1:2, 0:HF] + jnp.sum(rl * rl, axis=0, keepdims=True)
    st_ref[1:2, HF:F] = st_ref[1:2, HF:F] + jnp.sum(rr * rr, axis=0, keepdims=True)
    for g in range(NG):
        mg = bat == g
        ps_ref[g:g + 1, 0:HF] = ps_ref[g:g + 1, 0:HF] + jnp.sum(
            jnp.where(mg, rl, 0.0), axis=0, keepdims=True)
        ps_ref[g:g + 1, HF:F] = ps_ref[g:g + 1, HF:F] + jnp.sum(
            jnp.where(mg, rr, 0.0), axis=0, keepdims=True)
        pm_ref[g:g + 1, 0:HF] = jnp.maximum(
            pm_ref[g:g + 1, 0:HF],
            jnp.max(jnp.where(mg, rl, -jnp.inf), axis=0, keepdims=True))
        pm_ref[g:g + 1, HF:F] = jnp.maximum(
            pm_ref[g:g + 1, HF:F],
            jnp.max(jnp.where(mg, rr, -jnp.inf), axis=0, keepdims=True))
        pc_ref[g:g + 1, :] = pc_ref[g:g + 1, :] + jnp.sum(
            jnp.where(mg, 1.0, 0.0), axis=0, keepdims=True)


def _final_body(st_ref, ps_ref, pm_ref, pc_ref, g_ref, b_ref, wt_ref, lb_ref,
                cat_ref, out_ref):
    m2 = st_ref[0:1, :] / N
    v2 = st_ref[1:2, :] / N - m2 * m2
    sv = g_ref[...] * lax.rsqrt(v2 + EPS)
    tv = b_ref[...] - m2 * sv
    cnt = pc_ref[...]
    s_h = ps_ref[...] * sv + cnt * tv
    mx_h = pm_ref[...] * sv + tv
    mean_h = s_h / jnp.maximum(cnt, 1.0)
    cat_ref[:, 0:F] = s_h
    cat_ref[:, F:2 * F] = mean_h
    cat_ref[:, 2 * F:3 * F] = mx_h
    wt = wt_ref[...]
    out_ref[...] = (jnp.dot(s_h, wt[0:F], preferred_element_type=f32, precision=lax.Precision.HIGHEST)
                    + jnp.dot(mean_h, wt[F:2 * F], preferred_element_type=f32, precision=lax.Precision.HIGHEST)
                    + jnp.dot(mx_h, wt[2 * F:3 * F], preferred_element_type=f32, precision=lax.Precision.HIGHEST)
                    + lb_ref[...])


def _rowspec(width):
    return pl.BlockSpec((RB, width), lambda i: (i, 0))


def _fullspec(shape):
    return pl.BlockSpec(shape, lambda i: tuple(0 for _ in shape))


_SPLITSPEC = pl.BlockSpec((NC, RB, HF), lambda i: (0, i, 0))
_DEGSPEC = pl.BlockSpec((NC, RB, 16), lambda i: (0, i, 0))


def _step1(xp, B1, c1r, degp):
    return pl.pallas_call(
        _step1_body,
        grid=(GRID,),
        in_specs=[_rowspec(IN_F), _fullspec((IN_F, 192)), _fullspec((1, 192)),
                  _DEGSPEC],
        out_specs=[_SPLITSPEC, _SPLITSPEC, _rowspec(HF), _rowspec(F)],
        out_shape=[jax.ShapeDtypeStruct((NC, NP, HF), f32),
                   jax.ShapeDtypeStruct((NC, NP, HF), f32),
                   jax.ShapeDtypeStruct((NP, HF), f32),
                   jax.ShapeDtypeStruct((NP, F), f32)],
    )(xp, B1, c1r, degp)


def _step5(d0, degp, sp):
    return pl.pallas_call(
        _step5_body,
        grid=(GRID,),
        in_specs=[_rowspec(F), _DEGSPEC, _SPLITSPEC],
        out_specs=[_rowspec(F), _fullspec((8, F))],
        out_shape=[jax.ShapeDtypeStruct((NP, F), f32),
                   jax.ShapeDtypeStruct((8, F), f32)],
    )(d0, degp, sp)


def _step6(r, B2, b2r, degp, st, g, b):
    return pl.pallas_call(
        _step6_body,
        grid=(GRID,),
        in_specs=[_rowspec(F), _fullspec((F, 192)), _fullspec((1, 192)),
                  _DEGSPEC, _fullspec((8, F)), _fullspec((1, F)),
                  _fullspec((1, F))],
        out_specs=[_SPLITSPEC, _SPLITSPEC, _rowspec(F)],
        out_shape=[jax.ShapeDtypeStruct((NC, NP, HF), f32),
                   jax.ShapeDtypeStruct((NC, NP, HF), f32),
                   jax.ShapeDtypeStruct((NP, F), f32)],
    )(r, B2, b2r, degp, st, g, b)


def _step10(d2o, degp, sp, batp):
    return pl.pallas_call(
        _step10_body,
        grid=(GRID,),
        in_specs=[_rowspec(F), _DEGSPEC, _SPLITSPEC, _rowspec(1)],
        out_specs=[_fullspec((8, F))] * 4,
        out_shape=[jax.ShapeDtypeStruct((8, F), f32)] * 4,
    )(d2o, degp, sp, batp)


def _final(st, ps, pm, pc, g, b, wt, lb):
    def fs(shape):
        return pl.BlockSpec(shape, lambda: tuple(0 for _ in shape))
    return pl.pallas_call(
        _final_body,
        in_specs=[
            fs((8, F)), fs((8, F)), fs((8, F)),
            fs((8, F)), fs((1, F)), fs((1, F)),
            fs((192, 32)), fs((1, 32))],
        out_specs=[fs((8, 192)), fs((8, 32))],
        out_shape=[jax.ShapeDtypeStruct((NG, 192), f32),
                   jax.ShapeDtypeStruct((NG, 32), f32)],
    )(st, ps, pm, pc, g, b, wt, lb)


@jax.jit
def kernel(x, edge_index, batch, sparse_mask, sm_weight, sm_bias,
           conv1_W, conv1_b, bn1_g, bn1_b, conv2_W, conv2_b, bn2_g, bn2_b,
           lin_W, lin_b):
    i32 = jnp.int32
    cols = jnp.arange(IN_F, dtype=jnp.int32)
    oh_in = (sparse_mask[:, 0:1] == cols[None, :]).astype(f32)
    oh_out = (sparse_mask[:, 1:2] == cols[None, :]).astype(f32) * sm_weight[:, None]
    M = jnp.dot(oh_in.T, oh_out, precision=lax.Precision.HIGHEST)
    Wc1 = jnp.concatenate([2.0 * conv1_W[2], conv1_W[1], conv1_W[0] - conv1_W[2]], axis=1)
    B1 = jnp.dot(M, Wc1, precision=lax.Precision.HIGHEST)
    c1r = (jnp.dot(sm_bias, Wc1, precision=lax.Precision.HIGHEST) + jnp.concatenate(
        [jnp.zeros((F,), f32), jnp.zeros((F,), f32), conv1_b]))[None, :]
    B2 = jnp.concatenate([2.0 * conv2_W[2], conv2_W[1], conv2_W[0] - conv2_W[2]], axis=1)
    b2r = jnp.concatenate([jnp.zeros((F,), f32), jnp.zeros((F,), f32), conv2_b])[None, :]
    linWT = lin_W.T
    lbr = lin_b[None, :]
    g1 = bn1_g[None, :]; b1 = bn1_b[None, :]
    g2 = bn2_g[None, :]; b2 = bn2_b[None, :]

    xp = jnp.pad(x, ((0, NP - N), (0, 0)))
    batp = jnp.concatenate([batch, jnp.full((NP - N,), NG, i32)]).reshape(NP, 1)
    srcp = jnp.concatenate([edge_index[0], jnp.full((EP - E,), N, i32)])
    dstp = jnp.concatenate([edge_index[1], jnp.full((EP - E,), N, i32)])
    srci_d = srcp.reshape(NC * NS, EPT_D // 128, 128)
    srci = srcp.reshape(NS, EPT // 128, 128)
    dsti = dstp.reshape(NS, EPT // 128, 128)
    z16 = jnp.zeros((RPS, 16), f32)
    ones16 = jnp.ones((128, 16), f32)

    degp = _deg_pass(ones16, z16, srci_d)
    ga, cm1, d2b, d0 = _step1(xp, B1, c1r, degp)
    sc, _ = _conv_pass(ga, cm1, d2b, srci, dsti)
    r, st1 = _step5(d0, degp, sc)
    ga2, cm2, d2o = _step6(r, B2, b2r, degp, st1, g1, b1)
    sc2, _ = _conv_pass(ga2, cm2, d2b, srci, dsti)
    st2, ps, pm, pc = _step10(d2o, degp, sc2, batp)
    cat, out = _final(st2, ps, pm, pc, g2, b2, linWT, lbr)
    return cat, out


# speedup vs baseline: 12.1965x; 1.0186x over previous
import functools

import jax
import jax.numpy as jnp
from jax import lax
from jax.experimental import pallas as pl
from jax.experimental.pallas import tpu as pltpu
from jax.experimental.pallas import tpu_sc as plsc

N = 10000
E = 320000
IN_F = 128
F = 64
HF = 32
NG = 8
NP = 10240
EP = 327680
NC, NS = 2, 16
EPT = EP // NS
ROWS_IT = 8
CH = ROWS_IT * 128
N_IT = EPT // CH
RPS = NP // NS
EPT_D = EP // (NC * NS)
N_IT_D = EPT_D // CH
RB = 256
GRID = NP // RB
EPS = 1e-5
f32 = jnp.float32


def _sc_mesh():
    return plsc.VectorSubcoreMesh(core_axis_name="c", subcore_axis_name="s")


def _deg_body(ones_hbm, z_hbm, src_hbm, out_hbm, srcv, onesv, stage, acc, sem):
    c = lax.axis_index("c")
    s = lax.axis_index("s")
    w = c * NS + s
    pltpu.sync_copy(z_hbm, stage)
    pltpu.sync_copy(stage, acc.at[pl.ds(s * RPS, RPS)])
    pltpu.sync_copy(ones_hbm, onesv)
    plsc.subcore_barrier()

    @pl.loop(0, N_IT_D)
    def _(it):
        pltpu.sync_copy(src_hbm.at[w, pl.ds(it * ROWS_IT, ROWS_IT)], srcv)
        for j in range(ROWS_IT):
            pltpu.sync_copy(onesv, acc.at[srcv.at[j]], add=True)

    plsc.subcore_barrier()
    pltpu.sync_copy(acc.at[pl.ds(s * RPS, RPS)], stage)
    pltpu.sync_copy(stage, out_hbm.at[c, pl.ds(s * RPS, RPS)])


@jax.jit
def _deg_pass(ones16, z16, srci):
    kern = pl.kernel(
        _deg_body,
        out_type=jax.ShapeDtypeStruct((NC, NP, 16), f32),
        mesh=_sc_mesh(),
        compiler_params=pltpu.CompilerParams(use_tc_tiling_on_sc=False),
        scratch_types=[
            pltpu.VMEM((ROWS_IT, 128), jnp.int32),
            pltpu.VMEM((128, 16), f32),
            pltpu.VMEM((RPS, 16), f32),
            pltpu.VMEM_SHARED((NP, 16), f32),
            pltpu.SemaphoreType.DMA,
        ],
    )
    return kern(ones16, z16, srci)


HS = RPS // 4


def _conv_body(tab_hbm, cm_hbm, d2b_hbm, src_hbm, dst_hbm,
               out_hbm, gc_hbm,
               srcv0, dstv0, srcv1, dstv1, rows0, rows1, stage,
               accv, cmv, d2v, acc, gsem0, gsem1, ssem):
    c = lax.axis_index("c")
    s = lax.axis_index("s")
    srcv = (srcv0, srcv1)
    dstv = (dstv0, dstv1)
    rows = (rows0, rows1)
    gsem = (gsem0, gsem1)
    zv = jnp.zeros((16,), f32)

    @pl.loop(0, RPS)
    def _(rr):
        stage[rr, pl.ds(0, 16)] = zv
        stage[rr, pl.ds(16, 16)] = zv

    pltpu.sync_copy(stage, acc.at[pl.ds(s * RPS, RPS)])

    def scatter_sweep(t_ref):
        def load_idx(slot, itv):
            pltpu.sync_copy(src_hbm.at[s, pl.ds(itv * ROWS_IT, ROWS_IT)],
                            srcv[slot])
            pltpu.sync_copy(dst_hbm.at[s, pl.ds(itv * ROWS_IT, ROWS_IT)],
                            dstv[slot])

        def fire_gathers(slot):
            for j in range(ROWS_IT):
                pltpu.async_copy(t_ref.at[srcv[slot].at[j]],
                                 rows[slot].at[pl.ds(j * 128, 128)],
                                 gsem[slot])

        def wait_gathers(slot):
            for j in range(ROWS_IT):
                pltpu.make_async_copy(t_ref.at[srcv[slot].at[j]],
                                      rows[slot].at[pl.ds(j * 128, 128)],
                                      gsem[slot]).wait()

        load_idx(0, 0)
        fire_gathers(0)

        @pl.loop(0, N_IT, step=2)
        def _(it):
            for b in range(2):
                cur = it + b
                nb = 1 - b

                @pl.when(cur + 1 < N_IT)
                def _():
                    load_idx(nb, cur + 1)
                    fire_gathers(nb)

                wait_gathers(b)
                cps = []
                for j in range(ROWS_IT):
                    cps.append(pltpu.async_copy(
                        rows[b].at[pl.ds(j * 128, 128)],
                        acc.at[dstv[b].at[j]], ssem, add=True))
                for cp in cps:
                    cp.wait()

    plsc.subcore_barrier()
    scatter_sweep(tab_hbm.at[c])
    plsc.subcore_barrier()

    for half in range(4):
        base = s * RPS + half * HS
        pltpu.sync_copy(acc.at[pl.ds(base, HS)], accv)
        pltpu.sync_copy(cm_hbm.at[c, pl.ds(base, HS)], cmv)
        pltpu.sync_copy(d2b_hbm.at[pl.ds(base, HS)], d2v)

        @pl.loop(0, HS)
        def _(rr):
            for k in range(2):
                sl = pl.ds(k * 16, 16)
                cmv[rr, sl] = cmv[rr, sl] - d2v[rr, sl] * accv[rr, sl]

        pltpu.sync_copy(cmv, gc_hbm.at[c, pl.ds(base, HS)])

    pltpu.sync_copy(stage, acc.at[pl.ds(s * RPS, RPS)])
    plsc.subcore_barrier()
    scatter_sweep(gc_hbm.at[c])
    plsc.subcore_barrier()
    pltpu.sync_copy(acc.at[pl.ds(s * RPS, RPS)], stage)
    pltpu.sync_copy(stage, out_hbm.at[c, pl.ds(s * RPS, RPS)])


@jax.jit
def _conv_pass(tab, cm, d2b, srci, dsti):
    kern = pl.kernel(
        _conv_body,
        out_type=[jax.ShapeDtypeStruct((NC, NP, HF), f32),
                  jax.ShapeDtypeStruct((NC, NP, HF), f32)],
        mesh=_sc_mesh(),
        compiler_params=pltpu.CompilerParams(use_tc_tiling_on_sc=False),
        scratch_types=[
            pltpu.VMEM((ROWS_IT, 128), jnp.int32),
            pltpu.VMEM((ROWS_IT, 128), jnp.int32),
            pltpu.VMEM((ROWS_IT, 128), jnp.int32),
            pltpu.VMEM((ROWS_IT, 128), jnp.int32),
            pltpu.VMEM((CH, HF), f32),
            pltpu.VMEM((CH, HF), f32),
            pltpu.VMEM((RPS, HF), f32),
            pltpu.VMEM((HS, HF), f32),
            pltpu.VMEM((HS, HF), f32),
            pltpu.VMEM((HS, HF), f32),
            pltpu.VMEM_SHARED((NP, HF), f32),
            pltpu.SemaphoreType.DMA,
            pltpu.SemaphoreType.DMA,
            pltpu.SemaphoreType.DMA,
        ],
    )
    return kern(tab, cm, d2b, srci, dsti)


def _dis(degp):
    deg = degp[0, :, 0:1] + degp[1, :, 0:1]
    return jnp.where(deg > 0, lax.rsqrt(jnp.maximum(deg, 1.0)), 0.0)


def _valid(i):
    row = lax.broadcasted_iota(jnp.int32, (RB, 1), 0) + i * RB
    return row < N


def _step1_body(x_ref, b1_ref, c1_ref, degp_ref, ga_ref, cm_ref, d2b_ref,
                d0_ref):
    i = pl.program_id(0)
    d = _dis(degp_ref[...])
    v = _valid(i)
    G = jnp.dot(x_ref[...], b1_ref[...], preferred_element_type=f32, precision=lax.Precision.HIGHEST) + c1_ref[...]
    ga_ref[0] = jnp.where(v, d * G[:, 0:HF], 0.0)
    ga_ref[1] = jnp.where(v, d * G[:, HF:F], 0.0)
    cm_ref[0] = jnp.where(v, d * G[:, F:F + HF], 0.0)
    cm_ref[1] = jnp.where(v, d * G[:, F + HF:2 * F], 0.0)
    d2b_ref[...] = jnp.broadcast_to(d * d, (RB, HF))
    d0_ref[...] = jnp.where(v, G[:, 2 * F:3 * F], 0.0)


def _step5_body(d0_ref, degp_ref, sp_ref, r_ref, st_ref):
    i = pl.program_id(0)
    d = _dis(degp_ref[...])
    v = _valid(i)
    d0 = d0_ref[...]
    rl = jnp.where(v, jnp.maximum(d0[:, 0:HF] - d * sp_ref[0], 0.0), 0.0)
    rr = jnp.where(v, jnp.maximum(d0[:, HF:F] - d * sp_ref[1], 0.0), 0.0)
    r_ref[:, 0:HF] = rl
    r_ref[:, HF:F] = rr

    @pl.when(i == 0)
    def _():
        st_ref[...] = jnp.zeros((8, F), f32)

    st_ref[0:1, 0:HF] = st_ref[0:1, 0:HF] + jnp.sum(rl, axis=0, keepdims=True)
    st_ref[0:1, HF:F] = st_ref[0:1, HF:F] + jnp.sum(rr, axis=0, keepdims=True)
    st_ref[1:2, 0:HF] = st_ref[1:2, 0:HF] + jnp.sum(rl * rl, axis=0, keepdims=True)
    st_ref[1:2, HF:F] = st_ref[1:2, HF:F] + jnp.sum(rr * rr, axis=0, keepdims=True)


def _step6_body(r_ref, b2_ref, b2r_ref, degp_ref, st_ref, g_ref, b_ref,
                ga2_ref, c2d_ref, d2o_ref):
    i = pl.program_id(0)
    d = _dis(degp_ref[...])
    v = _valid(i)
    m = st_ref[0:1, :] / N
    var = st_ref[1:2, :] / N - m * m
    sv = g_ref[...] * lax.rsqrt(var + EPS)
    tv = b_ref[...] - m * sv
    h1 = jnp.where(v, r_ref[...] * sv + tv, 0.0)
    G2 = jnp.dot(h1, b2_ref[...], preferred_element_type=f32, precision=lax.Precision.HIGHEST) + b2r_ref[...]
    ga2_ref[0] = jnp.where(v, d * G2[:, 0:HF], 0.0)
    ga2_ref[1] = jnp.where(v, d * G2[:, HF:F], 0.0)
    c2d_ref[0] = jnp.where(v, d * G2[:, F:F + HF], 0.0)
    c2d_ref[1] = jnp.where(v, d * G2[:, F + HF:2 * F], 0.0)
    d2o_ref[...] = jnp.where(v, G2[:, 2 * F:3 * F], 0.0)


def _step10_body(d2o_ref, degp_ref, sp_ref, bat_ref, g_ref, b_ref,
                 wt_ref, lb_ref,
                 st_ref, ps_ref, pm_ref, pc_ref, cat_ref, out_ref):
    i = pl.program_id(0)
    d = _dis(degp_ref[...])
    v = _valid(i)
    d2o = d2o_ref[...]
    rl = jnp.where(v, jnp.maximum(d2o[:, 0:HF] - d * sp_ref[0], 0.0), 0.0)
    rr = jnp.where(v, jnp.maximum(d2o[:, HF:F] - d * sp_ref[1], 0.0), 0.0)
    bat = bat_ref[...]

    @pl.when(i == 0)
    def _():
        st_ref[...] = jnp.zeros((8, F), f32)
        ps_ref[...] = jnp.zeros((8, F), f32)
        pm_ref[...] = jnp.full((8, F), -jnp.inf, f32)
        pc_ref[...] = jnp.zeros((8, F), f32)

    st_ref[0:1, 0:HF] = st_ref[0:1, 0:HF] + jnp.sum(rl, axis=0, keepdims=True)
    st_ref[0:1, HF:F] = st_ref[0:1, HF:F] + jnp.sum(rr, axis=0, keepdims=True)
    st_ref[1:2, 0:HF] = st_ref[1:2, 0:HF] + jnp.sum(rl * rl, axis=0, keepdims=True)
    st_ref[1:2, HF:F] = st_ref[1:2, HF:F] + jnp.sum(rr * rr, axis=0, keepdims=True)
    for g in range(NG):
        mg = bat == g
        ps_ref[g:g + 1, 0:HF] = ps_ref[g:g + 1, 0:HF] + jnp.sum(
            jnp.where(mg, rl, 0.0), axis=0, keepdims=True)
        ps_ref[g:g + 1, HF:F] = ps_ref[g:g + 1, HF:F] + jnp.sum(
            jnp.where(mg, rr, 0.0), axis=0, keepdims=True)
        pm_ref[g:g + 1, 0:HF] = jnp.maximum(
            pm_ref[g:g + 1, 0:HF],
            jnp.max(jnp.where(mg, rl, -jnp.inf), axis=0, keepdims=True))
        pm_ref[g:g + 1, HF:F] = jnp.maximum(
            pm_ref[g:g + 1, HF:F],
            jnp.max(jnp.where(mg, rr, -jnp.inf), axis=0, keepdims=True))
        pc_ref[g:g + 1, :] = pc_ref[g:g + 1, :] + jnp.sum(
            jnp.where(mg, 1.0, 0.0), axis=0, keepdims=True)

    @pl.when(i == GRID - 1)
    def _():
        m2 = st_ref[0:1, :] / N
        v2 = st_ref[1:2, :] / N - m2 * m2
        sv = g_ref[...] * lax.rsqrt(v2 + EPS)
        tv = b_ref[...] - m2 * sv
        cnt = pc_ref[...]
        s_h = ps_ref[...] * sv + cnt * tv
        mx_h = pm_ref[...] * sv + tv
        mean_h = s_h / jnp.maximum(cnt, 1.0)
        cat_ref[:, 0:F] = s_h
        cat_ref[:, F:2 * F] = mean_h
        cat_ref[:, 2 * F:3 * F] = mx_h
        wt = wt_ref[...]
        out_ref[...] = (jnp.dot(s_h, wt[0:F], preferred_element_type=f32, precision=lax.Precision.HIGHEST)
                        + jnp.dot(mean_h, wt[F:2 * F], preferred_element_type=f32, precision=lax.Precision.HIGHEST)
                        + jnp.dot(mx_h, wt[2 * F:3 * F], preferred_element_type=f32, precision=lax.Precision.HIGHEST)
                        + lb_ref[...])


def _rowspec(width):
    return pl.BlockSpec((RB, width), lambda i: (i, 0))


def _fullspec(shape):
    return pl.BlockSpec(shape, lambda i: tuple(0 for _ in shape))


_SPLITSPEC = pl.BlockSpec((NC, RB, HF), lambda i: (0, i, 0))
_DEGSPEC = pl.BlockSpec((NC, RB, 16), lambda i: (0, i, 0))


def _step1(xp, B1, c1r, degp):
    return pl.pallas_call(
        _step1_body,
        grid=(GRID,),
        in_specs=[_rowspec(IN_F), _fullspec((IN_F, 192)), _fullspec((1, 192)),
                  _DEGSPEC],
        out_specs=[_SPLITSPEC, _SPLITSPEC, _rowspec(HF), _rowspec(F)],
        out_shape=[jax.ShapeDtypeStruct((NC, NP, HF), f32),
                   jax.ShapeDtypeStruct((NC, NP, HF), f32),
                   jax.ShapeDtypeStruct((NP, HF), f32),
                   jax.ShapeDtypeStruct((NP, F), f32)],
    )(xp, B1, c1r, degp)


def _step5(d0, degp, sp):
    return pl.pallas_call(
        _step5_body,
        grid=(GRID,),
        in_specs=[_rowspec(F), _DEGSPEC, _SPLITSPEC],
        out_specs=[_rowspec(F), _fullspec((8, F))],
        out_shape=[jax.ShapeDtypeStruct((NP, F), f32),
                   jax.ShapeDtypeStruct((8, F), f32)],
    )(d0, degp, sp)


def _step6(r, B2, b2r, degp, st, g, b):
    return pl.pallas_call(
        _step6_body,
        grid=(GRID,),
        in_specs=[_rowspec(F), _fullspec((F, 192)), _fullspec((1, 192)),
                  _DEGSPEC, _fullspec((8, F)), _fullspec((1, F)),
                  _fullspec((1, F))],
        out_specs=[_SPLITSPEC, _SPLITSPEC, _rowspec(F)],
        out_shape=[jax.ShapeDtypeStruct((NC, NP, HF), f32),
                   jax.ShapeDtypeStruct((NC, NP, HF), f32),
                   jax.ShapeDtypeStruct((NP, F), f32)],
    )(r, B2, b2r, degp, st, g, b)


def _step10(d2o, degp, sp, batp, g, b, wt, lb):
    return pl.pallas_call(
        _step10_body,
        grid=(GRID,),
        in_specs=[_rowspec(F), _DEGSPEC, _SPLITSPEC, _rowspec(1),
                  _fullspec((1, F)), _fullspec((1, F)),
                  _fullspec((192, 32)), _fullspec((1, 32))],
        out_specs=[_fullspec((8, F))] * 4 + [_fullspec((8, 192)),
                                            _fullspec((8, 32))],
        out_shape=[jax.ShapeDtypeStruct((8, F), f32)] * 4
        + [jax.ShapeDtypeStruct((NG, 192), f32),
           jax.ShapeDtypeStruct((NG, 32), f32)],
    )(d2o, degp, sp, batp, g, b, wt, lb)


@jax.jit
def kernel(x, edge_index, batch, sparse_mask, sm_weight, sm_bias,
           conv1_W, conv1_b, bn1_g, bn1_b, conv2_W, conv2_b, bn2_g, bn2_b,
           lin_W, lin_b):
    i32 = jnp.int32
    cols = jnp.arange(IN_F, dtype=jnp.int32)
    oh_in = (sparse_mask[:, 0:1] == cols[None, :]).astype(f32)
    oh_out = (sparse_mask[:, 1:2] == cols[None, :]).astype(f32) * sm_weight[:, None]
    M = jnp.dot(oh_in.T, oh_out, precision=lax.Precision.HIGHEST)
    Wc1 = jnp.concatenate([2.0 * conv1_W[2], conv1_W[1], conv1_W[0] - conv1_W[2]], axis=1)
    B1 = jnp.dot(M, Wc1, precision=lax.Precision.HIGHEST)
    c1r = (jnp.dot(sm_bias, Wc1, precision=lax.Precision.HIGHEST) + jnp.concatenate(
        [jnp.zeros((F,), f32), jnp.zeros((F,), f32), conv1_b]))[None, :]
    B2 = jnp.concatenate([2.0 * conv2_W[2], conv2_W[1], conv2_W[0] - conv2_W[2]], axis=1)
    b2r = jnp.concatenate([jnp.zeros((F,), f32), jnp.zeros((F,), f32), conv2_b])[None, :]
    linWT = lin_W.T
    lbr = lin_b[None, :]
    g1 = bn1_g[None, :]; b1 = bn1_b[None, :]
    g2 = bn2_g[None, :]; b2 = bn2_b[None, :]

    xp = x
    batp = jnp.concatenate([batch, jnp.full((NP - N,), NG, i32)]).reshape(NP, 1)
    srcp = jnp.concatenate([edge_index[0], jnp.full((EP - E,), N, i32)])
    dstp = jnp.concatenate([edge_index[1], jnp.full((EP - E,), N, i32)])
    srci_d = srcp.reshape(NC * NS, EPT_D // 128, 128)
    srci = srcp.reshape(NS, EPT // 128, 128)
    dsti = dstp.reshape(NS, EPT // 128, 128)
    z16 = jnp.zeros((RPS, 16), f32)
    ones16 = jnp.ones((128, 16), f32)

    degp = _deg_pass(ones16, z16, srci_d)
    ga, cm1, d2b, d0 = _step1(xp, B1, c1r, degp)
    sc, _ = _conv_pass(ga, cm1, d2b, srci, dsti)
    r, st1 = _step5(d0, degp, sc)
    ga2, cm2, d2o = _step6(r, B2, b2r, degp, st1, g1, b1)
    sc2, _ = _conv_pass(ga2, cm2, d2b, srci, dsti)
    _, _, _, _, cat, out = _step10(d2o, degp, sc2, batp, g2, b2, linWT, lbr)
    return cat, out


# speedup vs baseline: 12.1978x; 1.0001x over previous
import functools

import jax
import jax.numpy as jnp
from jax import lax
from jax.experimental import pallas as pl
from jax.experimental.pallas import tpu as pltpu
from jax.experimental.pallas import tpu_sc as plsc

N = 10000
E = 320000
IN_F = 128
F = 64
HF = 32
NG = 8
NP = 10240
EP = 327680
NC, NS = 2, 16
EPT = EP // NS
ROWS_IT = 8
CH = ROWS_IT * 128
N_IT = EPT // CH
RPS = NP // NS
EPT_D = EP // (NC * NS)
N_IT_D = EPT_D // CH
RB = 256
GRID = NP // RB
EPS = 1e-5
f32 = jnp.float32


def _sc_mesh():
    return plsc.VectorSubcoreMesh(core_axis_name="c", subcore_axis_name="s")


def _deg_body(ones_hbm, z_hbm, src_hbm, out_hbm, srcv, onesv, stage, acc, sem):
    c = lax.axis_index("c")
    s = lax.axis_index("s")
    w = c * NS + s
    pltpu.sync_copy(z_hbm, stage)
    pltpu.sync_copy(stage, acc.at[pl.ds(s * RPS, RPS)])
    pltpu.sync_copy(ones_hbm, onesv)
    plsc.subcore_barrier()

    @pl.loop(0, N_IT_D)
    def _(it):
        pltpu.sync_copy(src_hbm.at[w, pl.ds(it * ROWS_IT, ROWS_IT)], srcv)
        for j in range(ROWS_IT):
            pltpu.sync_copy(onesv, acc.at[srcv.at[j]], add=True)

    plsc.subcore_barrier()
    pltpu.sync_copy(acc.at[pl.ds(s * RPS, RPS)], stage)
    pltpu.sync_copy(stage, out_hbm.at[c, pl.ds(s * RPS, RPS)])


@jax.jit
def _deg_pass(ones16, z16, srci):
    kern = pl.kernel(
        _deg_body,
        out_type=jax.ShapeDtypeStruct((NC, NP, 16), f32),
        mesh=_sc_mesh(),
        compiler_params=pltpu.CompilerParams(use_tc_tiling_on_sc=False),
        scratch_types=[
            pltpu.VMEM((ROWS_IT, 128), jnp.int32),
            pltpu.VMEM((128, 16), f32),
            pltpu.VMEM((RPS, 16), f32),
            pltpu.VMEM_SHARED((NP, 16), f32),
            pltpu.SemaphoreType.DMA,
        ],
    )
    return kern(ones16, z16, srci)


HS = RPS // 4


def _conv_body(tab_hbm, cm_hbm, d2b_hbm, src_hbm, dst_hbm,
               out_hbm, gc_hbm,
               srcv0, dstv0, srcv1, dstv1, rows0, rows1, stage,
               accv, cmv, d2v, acc, gsem0, gsem1, ssem):
    c = lax.axis_index("c")
    s = lax.axis_index("s")
    srcv = (srcv0, srcv1)
    dstv = (dstv0, dstv1)
    rows = (rows0, rows1)
    gsem = (gsem0, gsem1)
    zv = jnp.zeros((16,), f32)

    @pl.loop(0, RPS)
    def _(rr):
        stage[rr, pl.ds(0, 16)] = zv
        stage[rr, pl.ds(16, 16)] = zv

    pltpu.sync_copy(stage, acc.at[pl.ds(s * RPS, RPS)])

    def scatter_sweep(t_ref):
        def load_idx(slot, itv):
            pltpu.sync_copy(src_hbm.at[s, pl.ds(itv * ROWS_IT, ROWS_IT)],
                            srcv[slot])
            pltpu.sync_copy(dst_hbm.at[s, pl.ds(itv * ROWS_IT, ROWS_IT)],
                            dstv[slot])

        def fire_gathers(slot):
            for j in range(ROWS_IT):
                pltpu.async_copy(t_ref.at[srcv[slot].at[j]],
                                 rows[slot].at[pl.ds(j * 128, 128)],
                                 gsem[slot])

        def wait_gathers(slot):
            for j in range(ROWS_IT):
                pltpu.make_async_copy(t_ref.at[srcv[slot].at[j]],
                                      rows[slot].at[pl.ds(j * 128, 128)],
                                      gsem[slot]).wait()

        load_idx(0, 0)
        fire_gathers(0)

        @pl.loop(0, N_IT, step=2)
        def _(it):
            for b in range(2):
                cur = it + b
                nb = 1 - b

                @pl.when(cur + 1 < N_IT)
                def _():
                    load_idx(nb, cur + 1)
                    fire_gathers(nb)

                wait_gathers(b)
                cps = []
                for j in range(ROWS_IT):
                    cps.append(pltpu.async_copy(
                        rows[b].at[pl.ds(j * 128, 128)],
                        acc.at[dstv[b].at[j]], ssem, add=True))
                for cp in cps:
                    cp.wait()

    plsc.subcore_barrier()
    scatter_sweep(tab_hbm.at[c])
    plsc.subcore_barrier()

    for half in range(4):
        base = s * RPS + half * HS
        pltpu.sync_copy(acc.at[pl.ds(base, HS)], accv)
        pltpu.sync_copy(cm_hbm.at[c, pl.ds(base, HS)], cmv)
        pltpu.sync_copy(d2b_hbm.at[pl.ds(base, HS)], d2v)

        @pl.loop(0, HS)
        def _(rr):
            for k in range(2):
                sl = pl.ds(k * 16, 16)
                cmv[rr, sl] = cmv[rr, sl] - d2v[rr, sl] * accv[rr, sl]

        pltpu.sync_copy(cmv, gc_hbm.at[c, pl.ds(base, HS)])

    pltpu.sync_copy(stage, acc.at[pl.ds(s * RPS, RPS)])
    plsc.subcore_barrier()
    scatter_sweep(gc_hbm.at[c])
    plsc.subcore_barrier()
    pltpu.sync_copy(acc.at[pl.ds(s * RPS, RPS)],
                    out_hbm.at[c, pl.ds(s * RPS, RPS)])


@jax.jit
def _conv_pass(tab, cm, d2b, srci, dsti):
    kern = pl.kernel(
        _conv_body,
        out_type=[jax.ShapeDtypeStruct((NC, NP, HF), f32),
                  jax.ShapeDtypeStruct((NC, NP, HF), f32)],
        mesh=_sc_mesh(),
        compiler_params=pltpu.CompilerParams(use_tc_tiling_on_sc=False),
        scratch_types=[
            pltpu.VMEM((ROWS_IT, 128), jnp.int32),
            pltpu.VMEM((ROWS_IT, 128), jnp.int32),
            pltpu.VMEM((ROWS_IT, 128), jnp.int32),
            pltpu.VMEM((ROWS_IT, 128), jnp.int32),
            pltpu.VMEM((CH, HF), f32),
            pltpu.VMEM((CH, HF), f32),
            pltpu.VMEM((RPS, HF), f32),
            pltpu.VMEM((HS, HF), f32),
            pltpu.VMEM((HS, HF), f32),
            pltpu.VMEM((HS, HF), f32),
            pltpu.VMEM_SHARED((NP, HF), f32),
            pltpu.SemaphoreType.DMA,
            pltpu.SemaphoreType.DMA,
            pltpu.SemaphoreType.DMA,
        ],
    )
    return kern(tab, cm, d2b, srci, dsti)


def _dis(degp):
    deg = degp[0, :, 0:1] + degp[1, :, 0:1]
    return jnp.where(deg > 0, lax.rsqrt(jnp.maximum(deg, 1.0)), 0.0)


def _valid(i):
    row = lax.broadcasted_iota(jnp.int32, (RB, 1), 0) + i * RB
    return row < N


def _step1_body(x_ref, b1_ref, c1_ref, degp_ref, ga_ref, cm_ref, d2b_ref,
                d0_ref):
    i = pl.program_id(0)
    d = _dis(degp_ref[...])
    v = _valid(i)
    G = jnp.dot(x_ref[...], b1_ref[...], preferred_element_type=f32, precision=lax.Precision.HIGHEST) + c1_ref[...]
    ga_ref[0] = jnp.where(v, d * G[:, 0:HF], 0.0)
    ga_ref[1] = jnp.where(v, d * G[:, HF:F], 0.0)
    cm_ref[0] = jnp.where(v, d * G[:, F:F + HF], 0.0)
    cm_ref[1] = jnp.where(v, d * G[:, F + HF:2 * F], 0.0)
    d2b_ref[...] = jnp.broadcast_to(d * d, (RB, HF))
    d0_ref[...] = jnp.where(v, G[:, 2 * F:3 * F], 0.0)


def _step5_body(d0_ref, degp_ref, sp_ref, r_ref, st_ref):
    i = pl.program_id(0)
    d = _dis(degp_ref[...])
    v = _valid(i)
    d0 = d0_ref[...]
    rl = jnp.where(v, jnp.maximum(d0[:, 0:HF] - d * sp_ref[0], 0.0), 0.0)
    rr = jnp.where(v, jnp.maximum(d0[:, HF:F] - d * sp_ref[1], 0.0), 0.0)
    r_ref[:, 0:HF] = rl
    r_ref[:, HF:F] = rr

    @pl.when(i == 0)
    def _():
        st_ref[...] = jnp.zeros((8, F), f32)

    st_ref[0:1, 0:HF] = st_ref[0:1, 0:HF] + jnp.sum(rl, axis=0, keepdims=True)
    st_ref[0:1, HF:F] = st_ref[0:1, HF:F] + jnp.sum(rr, axis=0, keepdims=True)
    st_ref[1:2, 0:HF] = st_ref[1:2, 0:HF] + jnp.sum(rl * rl, axis=0, keepdims=True)
    st_ref[1:2, HF:F] = st_ref[1:2, HF:F] + jnp.sum(rr * rr, axis=0, keepdims=True)


def _step6_body(r_ref, b2_ref, b2r_ref, degp_ref, st_ref, g_ref, b_ref,
                ga2_ref, c2d_ref, d2o_ref):
    i = pl.program_id(0)
    d = _dis(degp_ref[...])
    v = _valid(i)
    m = st_ref[0:1, :] / N
    var = st_ref[1:2, :] / N - m * m
    sv = g_ref[...] * lax.rsqrt(var + EPS)
    tv = b_ref[...] - m * sv
    h1 = jnp.where(v, r_ref[...] * sv + tv, 0.0)
    G2 = jnp.dot(h1, b2_ref[...], preferred_element_type=f32, precision=lax.Precision.HIGHEST) + b2r_ref[...]
    ga2_ref[0] = jnp.where(v, d * G2[:, 0:HF], 0.0)
    ga2_ref[1] = jnp.where(v, d * G2[:, HF:F], 0.0)
    c2d_ref[0] = jnp.where(v, d * G2[:, F:F + HF], 0.0)
    c2d_ref[1] = jnp.where(v, d * G2[:, F + HF:2 * F], 0.0)
    d2o_ref[...] = jnp.where(v, G2[:, 2 * F:3 * F], 0.0)


def _step10_body(d2o_ref, degp_ref, sp_ref, bat_ref, g_ref, b_ref,
                 wt_ref, lb_ref,
                 st_ref, ps_ref, pm_ref, pc_ref, cat_ref, out_ref):
    i = pl.program_id(0)
    d = _dis(degp_ref[...])
    v = _valid(i)
    d2o = d2o_ref[...]
    rl = jnp.where(v, jnp.maximum(d2o[:, 0:HF] - d * sp_ref[0], 0.0), 0.0)
    rr = jnp.where(v, jnp.maximum(d2o[:, HF:F] - d * sp_ref[1], 0.0), 0.0)
    bat = bat_ref[...]

    @pl.when(i == 0)
    def _():
        st_ref[...] = jnp.zeros((8, F), f32)
        ps_ref[...] = jnp.zeros((8, F), f32)
        pm_ref[...] = jnp.full((8, F), -jnp.inf, f32)
        pc_ref[...] = jnp.zeros((8, F), f32)

    st_ref[0:1, 0:HF] = st_ref[0:1, 0:HF] + jnp.sum(rl, axis=0, keepdims=True)
    st_ref[0:1, HF:F] = st_ref[0:1, HF:F] + jnp.sum(rr, axis=0, keepdims=True)
    st_ref[1:2, 0:HF] = st_ref[1:2, 0:HF] + jnp.sum(rl * rl, axis=0, keepdims=True)
    st_ref[1:2, HF:F] = st_ref[1:2, HF:F] + jnp.sum(rr * rr, axis=0, keepdims=True)
    for g in range(NG):
        mg = bat == g
        ps_ref[g:g + 1, 0:HF] = ps_ref[g:g + 1, 0:HF] + jnp.sum(
            jnp.where(mg, rl, 0.0), axis=0, keepdims=True)
        ps_ref[g:g + 1, HF:F] = ps_ref[g:g + 1, HF:F] + jnp.sum(
            jnp.where(mg, rr, 0.0), axis=0, keepdims=True)
        pm_ref[g:g + 1, 0:HF] = jnp.maximum(
            pm_ref[g:g + 1, 0:HF],
            jnp.max(jnp.where(mg, rl, -jnp.inf), axis=0, keepdims=True))
        pm_ref[g:g + 1, HF:F] = jnp.maximum(
            pm_ref[g:g + 1, HF:F],
            jnp.max(jnp.where(mg, rr, -jnp.inf), axis=0, keepdims=True))
        pc_ref[g:g + 1, :] = pc_ref[g:g + 1, :] + jnp.sum(
            jnp.where(mg, 1.0, 0.0), axis=0, keepdims=True)

    @pl.when(i == GRID - 1)
    def _():
        m2 = st_ref[0:1, :] / N
        v2 = st_ref[1:2, :] / N - m2 * m2
        sv = g_ref[...] * lax.rsqrt(v2 + EPS)
        tv = b_ref[...] - m2 * sv
        cnt = pc_ref[...]
        s_h = ps_ref[...] * sv + cnt * tv
        mx_h = pm_ref[...] * sv + tv
        mean_h = s_h / jnp.maximum(cnt, 1.0)
        cat_ref[:, 0:F] = s_h
        cat_ref[:, F:2 * F] = mean_h
        cat_ref[:, 2 * F:3 * F] = mx_h
        wt = wt_ref[...]
        out_ref[...] = (jnp.dot(s_h, wt[0:F], preferred_element_type=f32, precision=lax.Precision.HIGHEST)
                        + jnp.dot(mean_h, wt[F:2 * F], preferred_element_type=f32, precision=lax.Precision.HIGHEST)
                        + jnp.dot(mx_h, wt[2 * F:3 * F], preferred_element_type=f32, precision=lax.Precision.HIGHEST)
                        + lb_ref[...])


def _rowspec(width):
    return pl.BlockSpec((RB, width), lambda i: (i, 0))


def _fullspec(shape):
    return pl.BlockSpec(shape, lambda i: tuple(0 for _ in shape))


_SPLITSPEC = pl.BlockSpec((NC, RB, HF), lambda i: (0, i, 0))
_DEGSPEC = pl.BlockSpec((NC, RB, 16), lambda i: (0, i, 0))


def _step1(xp, B1, c1r, degp):
    return pl.pallas_call(
        _step1_body,
        grid=(GRID,),
        in_specs=[_rowspec(IN_F), _fullspec((IN_F, 192)), _fullspec((1, 192)),
                  _DEGSPEC],
        out_specs=[_SPLITSPEC, _SPLITSPEC, _rowspec(HF), _rowspec(F)],
        out_shape=[jax.ShapeDtypeStruct((NC, NP, HF), f32),
                   jax.ShapeDtypeStruct((NC, NP, HF), f32),
                   jax.ShapeDtypeStruct((NP, HF), f32),
                   jax.ShapeDtypeStruct((NP, F), f32)],
    )(xp, B1, c1r, degp)


def _step5(d0, degp, sp):
    return pl.pallas_call(
        _step5_body,
        grid=(GRID,),
        in_specs=[_rowspec(F), _DEGSPEC, _SPLITSPEC],
        out_specs=[_rowspec(F), _fullspec((8, F))],
        out_shape=[jax.ShapeDtypeStruct((NP, F), f32),
                   jax.ShapeDtypeStruct((8, F), f32)],
    )(d0, degp, sp)


def _step6(r, B2, b2r, degp, st, g, b):
    return pl.pallas_call(
        _step6_body,
        grid=(GRID,),
        in_specs=[_rowspec(F), _fullspec((F, 192)), _fullspec((1, 192)),
                  _DEGSPEC, _fullspec((8, F)), _fullspec((1, F)),
                  _fullspec((1, F))],
        out_specs=[_SPLITSPEC, _SPLITSPEC, _rowspec(F)],
        out_shape=[jax.ShapeDtypeStruct((NC, NP, HF), f32),
                   jax.ShapeDtypeStruct((NC, NP, HF), f32),
                   jax.ShapeDtypeStruct((NP, F), f32)],
    )(r, B2, b2r, degp, st, g, b)


def _step10(d2o, degp, sp, batp, g, b, wt, lb):
    return pl.pallas_call(
        _step10_body,
        grid=(GRID,),
        in_specs=[_rowspec(F), _DEGSPEC, _SPLITSPEC, _rowspec(1),
                  _fullspec((1, F)), _fullspec((1, F)),
                  _fullspec((192, 32)), _fullspec((1, 32))],
        out_specs=[_fullspec((8, F))] * 4 + [_fullspec((8, 192)),
                                            _fullspec((8, 32))],
        out_shape=[jax.ShapeDtypeStruct((8, F), f32)] * 4
        + [jax.ShapeDtypeStruct((NG, 192), f32),
           jax.ShapeDtypeStruct((NG, 32), f32)],
    )(d2o, degp, sp, batp, g, b, wt, lb)


@jax.jit
def kernel(x, edge_index, batch, sparse_mask, sm_weight, sm_bias,
           conv1_W, conv1_b, bn1_g, bn1_b, conv2_W, conv2_b, bn2_g, bn2_b,
           lin_W, lin_b):
    i32 = jnp.int32
    cols = jnp.arange(IN_F, dtype=jnp.int32)
    oh_in = (sparse_mask[:, 0:1] == cols[None, :]).astype(f32)
    oh_out = (sparse_mask[:, 1:2] == cols[None, :]).astype(f32) * sm_weight[:, None]
    M = jnp.dot(oh_in.T, oh_out, precision=lax.Precision.HIGHEST)
    Wc1 = jnp.concatenate([2.0 * conv1_W[2], conv1_W[1], conv1_W[0] - conv1_W[2]], axis=1)
    B1 = jnp.dot(M, Wc1, precision=lax.Precision.HIGHEST)
    c1r = (jnp.dot(sm_bias, Wc1, precision=lax.Precision.HIGHEST) + jnp.concatenate(
        [jnp.zeros((F,), f32), jnp.zeros((F,), f32), conv1_b]))[None, :]
    B2 = jnp.concatenate([2.0 * conv2_W[2], conv2_W[1], conv2_W[0] - conv2_W[2]], axis=1)
    b2r = jnp.concatenate([jnp.zeros((F,), f32), jnp.zeros((F,), f32), conv2_b])[None, :]
    linWT = lin_W.T
    lbr = lin_b[None, :]
    g1 = bn1_g[None, :]; b1 = bn1_b[None, :]
    g2 = bn2_g[None, :]; b2 = bn2_b[None, :]

    xp = x
    batp = jnp.concatenate([batch, jnp.full((NP - N,), NG, i32)]).reshape(NP, 1)
    srcp = jnp.concatenate([edge_index[0], jnp.full((EP - E,), N, i32)])
    dstp = jnp.concatenate([edge_index[1], jnp.full((EP - E,), N, i32)])
    srci_d = srcp.reshape(NC * NS, EPT_D // 128, 128)
    srci = srcp.reshape(NS, EPT // 128, 128)
    dsti = dstp.reshape(NS, EPT // 128, 128)
    z16 = jnp.zeros((RPS, 16), f32)
    ones16 = jnp.ones((128, 16), f32)

    degp = _deg_pass(ones16, z16, srci_d)
    ga, cm1, d2b, d0 = _step1(xp, B1, c1r, degp)
    sc, _ = _conv_pass(ga, cm1, d2b, srci, dsti)
    r, st1 = _step5(d0, degp, sc)
    ga2, cm2, d2o = _step6(r, B2, b2r, degp, st1, g1, b1)
    sc2, _ = _conv_pass(ga2, cm2, d2b, srci, dsti)
    _, _, _, _, cat, out = _step10(d2o, degp, sc2, batp, g2, b2, linWT, lbr)
    return cat, out
